# Initial kernel scaffold; baseline (speedup 1.0000x reference)
import jax
import jax.numpy as jnp
from jax.experimental import pallas as pl


def kernel(x, edge_index, edge_type, W_emb, b_emb, ln_g, ln_b, w_base, w_spline, coeffs, attention, W_ih, W_hh, b_ih, b_hh):
    raise NotImplementedError("write your pallas kernel here")


# speedup vs baseline: 1.9096x; 1.9096x over previous
import functools

import numpy as np
import jax
import jax.numpy as jnp
from jax import lax
from jax.experimental import pallas as pl
from jax.experimental.pallas import tpu as pltpu
from jax.experimental.pallas import tpu_sc as plsc

_DEG = 3
_NB = 7
_KNOTS = [float(v) for v in np.linspace(-7.0, 7.0, _NB + _DEG + 1).astype(np.float32)]

_BE = 1000
_BN = 1000
_CK = 128
_ZR = 1000


def _sc_gather(h, src):
    n_nodes, H = h.shape
    E = src.shape[0]
    info = plsc.get_sparse_core_info()
    NC, NS = info.num_cores, info.num_subcores
    NW = NC * NS
    nch = E // _CK
    iters = (nch + NW - 1) // NW
    mesh = plsc.VectorSubcoreMesh(core_axis_name="c", subcore_axis_name="s")

    @functools.partial(
        pl.kernel,
        out_type=jax.ShapeDtypeStruct((E, H), jnp.float32),
        mesh=mesh,
        compiler_params=pltpu.CompilerParams(use_tc_tiling_on_sc=False),
        scratch_types=[
            pltpu.VMEM((_CK,), jnp.int32),
            pltpu.VMEM((_CK, H), jnp.float32),
            pltpu.SemaphoreType.DMA,
        ],
    )
    def gk(h_hbm, src_hbm, out_hbm, idx_v, rows_v, sem):
        wid = lax.axis_index("s") * NC + lax.axis_index("c")

        def body(j, carry):
            g = j * NW + wid

            @pl.when(g < nch)
            def _():
                base = pl.multiple_of(g * _CK, _CK)
                pltpu.sync_copy(src_hbm.at[pl.ds(base, _CK)], idx_v)
                pltpu.async_copy(h_hbm.at[idx_v], rows_v, sem).wait()
                pltpu.sync_copy(rows_v, out_hbm.at[pl.ds(base, _CK), :])

            return carry

        lax.fori_loop(0, iters, body, 0)

    return gk(h, src)


def _sc_scatter(val, dst, zeros_blk, n_nodes):
    E, H = val.shape
    info = plsc.get_sparse_core_info()
    NC, NS = info.num_cores, info.num_subcores
    NW = NC * NS
    nch = E // _CK
    iters = (nch + NW - 1) // NW
    NZ = n_nodes // _ZR
    mesh = plsc.VectorSubcoreMesh(core_axis_name="c", subcore_axis_name="s")

    @functools.partial(
        pl.kernel,
        out_type=jax.ShapeDtypeStruct((NC, n_nodes, H), jnp.float32),
        mesh=mesh,
        compiler_params=pltpu.CompilerParams(use_tc_tiling_on_sc=False),
        scratch_types=[
            pltpu.VMEM((_CK,), jnp.int32),
            pltpu.VMEM((_CK, H), jnp.float32),
            pltpu.VMEM_SHARED((n_nodes, H), jnp.float32),
        ],
    )
    def sk(val_hbm, dst_hbm, z_hbm, out_hbm, idx_v, rows_v, acc):
        c = lax.axis_index("c")
        s = lax.axis_index("s")
        wid = s * NC + c

        @pl.when(s < NZ)
        def _():
            off = pl.multiple_of(s * _ZR, 8)
            pltpu.sync_copy(z_hbm, acc.at[pl.ds(off, _ZR), :])

        plsc.subcore_barrier()

        def body(j, carry):
            g = j * NW + wid

            @pl.when(g < nch)
            def _():
                base = pl.multiple_of(g * _CK, _CK)
                pltpu.sync_copy(dst_hbm.at[pl.ds(base, _CK)], idx_v)
                pltpu.sync_copy(val_hbm.at[pl.ds(base, _CK), :], rows_v)
                pltpu.sync_copy(rows_v, acc.at[idx_v], add=True)

            return carry

        lax.fori_loop(0, iters, body, 0)
        plsc.subcore_barrier()

        @pl.when(s < NZ)
        def _():
            off = pl.multiple_of(s * _ZR, 8)
            pltpu.sync_copy(acc.at[pl.ds(off, _ZR), :], out_hbm.at[c, pl.ds(off, _ZR), :])

    return sk(val, dst, zeros_blk)


def _embed(x, W_emb, b_emb, ln_g, ln_b):
    n_nodes, D = x.shape
    H = W_emb.shape[0]
    G = n_nodes // _BN

    def body(x_ref, w_ref, b_ref, g_ref, bb_ref, out_ref):
        xv = x_ref[...]
        hm = lax.dot_general(xv, w_ref[...], (((1,), (1,)), ((), ())),
                             preferred_element_type=jnp.float32) + b_ref[...]
        mu = jnp.mean(hm, axis=1, keepdims=True)
        var = jnp.mean((hm - mu) ** 2, axis=1, keepdims=True)
        hn = (hm - mu) / jnp.sqrt(var + 1e-5) * g_ref[...] + bb_ref[...]
        out_ref[...] = jnp.maximum(hn, 0.0)

    return pl.pallas_call(
        body,
        grid=(G,),
        in_specs=[
            pl.BlockSpec((_BN, D), lambda i: (i, 0)),
            pl.BlockSpec((H, D), lambda i: (0, 0)),
            pl.BlockSpec((1, H), lambda i: (0, 0)),
            pl.BlockSpec((1, H), lambda i: (0, 0)),
            pl.BlockSpec((1, H), lambda i: (0, 0)),
        ],
        out_specs=pl.BlockSpec((_BN, H), lambda i: (i, 0)),
        out_shape=jax.ShapeDtypeStruct((n_nodes, H), jnp.float32),
    )(x, W_emb, b_emb.reshape(1, H), ln_g.reshape(1, H), ln_b.reshape(1, H))


def _bspline_tr(hs, et, coeffs_ref, wb_ref, ws_ref, R):
    t = _KNOTS
    base = hs * jax.nn.sigmoid(hs)
    B = [jnp.where((hs >= t[i]) & (hs < t[i + 1]), 1.0, 0.0) for i in range(_NB + _DEG)]
    for d in range(1, _DEG + 1):
        m = _NB + _DEG - d
        Bn = []
        for i in range(m):
            il = 1.0 / (t[i + d] - t[i])
            ir = 1.0 / (t[i + d + 1] - t[i + 1])
            Bn.append((hs - t[i]) * il * B[i] + (t[i + d + 1] - hs) * ir * B[i + 1])
        B = Bn
    oh = [et == r for r in range(R)]
    zcol = jnp.zeros_like(hs[:, :1])
    spline = jnp.zeros_like(hs)
    for n in range(_NB):
        ce_n = zcol
        for r in range(R):
            ce_n = jnp.where(oh[r], coeffs_ref[r, n], ce_n)
        spline = spline + B[n] * ce_n
    wb = zcol
    ws = zcol
    for r in range(R):
        wb = jnp.where(oh[r], wb_ref[r], wb)
        ws = jnp.where(oh[r], ws_ref[r], ws)
    return wb * base + ws * spline


def _edge_transform(hs, et3, coeffs, w_base, w_spline, attention):
    E, H = hs.shape
    G = E // _BE
    R = w_base.shape[0]

    def body(hs_ref, et_ref, coeffs_ref, wb_ref, ws_ref, att_ref, tr_ref, sc_ref):
        hs_v = hs_ref[...]
        et = et_ref[0]
        tr = _bspline_tr(hs_v, et, coeffs_ref, wb_ref, ws_ref, R)
        sc = jnp.sum(tr * att_ref[...], axis=1, keepdims=True)
        tr_ref[...] = tr
        sc_ref[0] = sc

    return pl.pallas_call(
        body,
        grid=(G,),
        in_specs=[
            pl.BlockSpec((_BE, H), lambda i: (i, 0)),
            pl.BlockSpec((1, _BE, 1), lambda i: (i, 0, 0)),
            pl.BlockSpec(memory_space=pltpu.SMEM),
            pl.BlockSpec(memory_space=pltpu.SMEM),
            pl.BlockSpec(memory_space=pltpu.SMEM),
            pl.BlockSpec((1, H), lambda i: (0, 0)),
        ],
        out_specs=[
            pl.BlockSpec((_BE, H), lambda i: (i, 0)),
            pl.BlockSpec((1, _BE, 1), lambda i: (i, 0, 0)),
        ],
        out_shape=[
            jax.ShapeDtypeStruct((E, H), jnp.float32),
            jax.ShapeDtypeStruct((G, _BE, 1), jnp.float32),
        ],
    )(hs, et3, coeffs, w_base, w_spline, attention.reshape(1, H))


def _softmax_stats(sc2, et2, R):
    def body(sc_ref, et_ref, m_ref, s_ref):
        sc = sc_ref[...]
        et = et_ref[...]
        for r in range(R):
            scm = jnp.where(et == r, sc, -1e30)
            mr = jnp.max(scm)
            m_ref[r] = mr
            s_ref[r] = jnp.sum(jnp.exp(scm - mr))

    return pl.pallas_call(
        body,
        out_specs=[
            pl.BlockSpec(memory_space=pltpu.SMEM),
            pl.BlockSpec(memory_space=pltpu.SMEM),
        ],
        out_shape=[
            jax.ShapeDtypeStruct((R,), jnp.float32),
            jax.ShapeDtypeStruct((R,), jnp.float32),
        ],
    )(sc2, et2)


def _scale(tr, sc3, et3, m, s):
    E, H = tr.shape
    G = E // _BE
    R = m.shape[0]

    def body(tr_ref, sc_ref, et_ref, m_ref, s_ref, val_ref, attn_ref):
        sc = sc_ref[0]
        et = et_ref[0]
        m_e = jnp.zeros_like(sc)
        s_e = jnp.ones_like(sc)
        for r in range(R):
            m_e = jnp.where(et == r, m_ref[r], m_e)
            s_e = jnp.where(et == r, s_ref[r], s_e)
        attn = jnp.exp(sc - m_e) / s_e
        val_ref[...] = tr_ref[...] * attn
        attn_ref[0] = attn

    return pl.pallas_call(
        body,
        grid=(G,),
        in_specs=[
            pl.BlockSpec((_BE, H), lambda i: (i, 0)),
            pl.BlockSpec((1, _BE, 1), lambda i: (i, 0, 0)),
            pl.BlockSpec((1, _BE, 1), lambda i: (i, 0, 0)),
            pl.BlockSpec(memory_space=pltpu.SMEM),
            pl.BlockSpec(memory_space=pltpu.SMEM),
        ],
        out_specs=[
            pl.BlockSpec((_BE, H), lambda i: (i, 0)),
            pl.BlockSpec((1, _BE, 1), lambda i: (i, 0, 0)),
        ],
        out_shape=[
            jax.ShapeDtypeStruct((E, H), jnp.float32),
            jax.ShapeDtypeStruct((G, _BE, 1), jnp.float32),
        ],
    )(tr, sc3, et3, m, s)


def _gru(msg2, h, W_ih, W_hh, b_ih, b_hh):
    n_nodes, H = h.shape
    NC = msg2.shape[0]
    G = n_nodes // _BN

    def body(msg_ref, h_ref, wih_ref, whh_ref, bih_ref, bhh_ref, out_ref):
        msg = msg_ref[0]
        for c in range(1, NC):
            msg = msg + msg_ref[c]
        hv = h_ref[...]
        gi = lax.dot_general(msg, wih_ref[...], (((1,), (1,)), ((), ())),
                             preferred_element_type=jnp.float32) + bih_ref[...]
        gh = lax.dot_general(hv, whh_ref[...], (((1,), (1,)), ((), ())),
                             preferred_element_type=jnp.float32) + bhh_ref[...]
        rg = jax.nn.sigmoid(gi[:, :H] + gh[:, :H])
        zg = jax.nn.sigmoid(gi[:, H:2 * H] + gh[:, H:2 * H])
        ng = jnp.tanh(gi[:, 2 * H:] + rg * gh[:, 2 * H:])
        out_ref[...] = (1.0 - zg) * ng + zg * hv

    return pl.pallas_call(
        body,
        grid=(G,),
        in_specs=[
            pl.BlockSpec((NC, _BN, H), lambda i: (0, i, 0)),
            pl.BlockSpec((_BN, H), lambda i: (i, 0)),
            pl.BlockSpec((3 * H, H), lambda i: (0, 0)),
            pl.BlockSpec((3 * H, H), lambda i: (0, 0)),
            pl.BlockSpec((1, 3 * H), lambda i: (0, 0)),
            pl.BlockSpec((1, 3 * H), lambda i: (0, 0)),
        ],
        out_specs=pl.BlockSpec((_BN, H), lambda i: (i, 0)),
        out_shape=jax.ShapeDtypeStruct((n_nodes, H), jnp.float32),
    )(msg2, h, W_ih, W_hh, b_ih.reshape(1, 3 * H), b_hh.reshape(1, 3 * H))


def kernel(x, edge_index, edge_type, W_emb, b_emb, ln_g, ln_b, w_base, w_spline,
           coeffs, attention, W_ih, W_hh, b_ih, b_hh):
    n_nodes, _ = x.shape
    H = W_emb.shape[0]
    E = edge_type.shape[0]
    R = w_base.shape[0]
    src = edge_index[0].astype(jnp.int32)
    dst = edge_index[1].astype(jnp.int32)
    et = edge_type.astype(jnp.int32)
    et3 = et.reshape(E // _BE, _BE, 1)
    et2 = et.reshape(E // 128, 128)
    zeros_blk = jnp.zeros((_ZR, H), jnp.float32)

    h = _embed(x, W_emb, b_emb, ln_g, ln_b)
    attns = []
    for _ in range(2):
        hs = _sc_gather(h, src)
        tr, sc3 = _edge_transform(hs, et3, coeffs, w_base, w_spline, attention)
        m, s = _softmax_stats(sc3.reshape(E // 128, 128), et2, R)
        val, attn3 = _scale(tr, sc3, et3, m, s)
        msg2 = _sc_scatter(val, dst, zeros_blk, n_nodes)
        h = _gru(msg2, h, W_ih, W_hh, b_ih, b_hh)
        attns.append(attn3.reshape(E))
    return h, jnp.stack(attns)


# speedup vs baseline: 2.0242x; 1.0600x over previous
import functools

import numpy as np
import jax
import jax.numpy as jnp
from jax import lax
from jax.experimental import pallas as pl
from jax.experimental.pallas import tpu as pltpu
from jax.experimental.pallas import tpu_sc as plsc

_DEG = 3
_NB = 7
_KNOTS = [float(v) for v in np.linspace(-7.0, 7.0, _NB + _DEG + 1).astype(np.float32)]

_BE = 1000
_BN = 1000
_CK = 128
_ZR = 1000


def _sc_gather(h, src):
    n_nodes, H = h.shape
    E = src.shape[0]
    info = plsc.get_sparse_core_info()
    NC, NS = info.num_cores, info.num_subcores
    NW = NC * NS
    nch = E // _CK
    iters = (nch + NW - 1) // NW
    mesh = plsc.VectorSubcoreMesh(core_axis_name="c", subcore_axis_name="s")

    @functools.partial(
        pl.kernel,
        out_type=jax.ShapeDtypeStruct((E, H), jnp.float32),
        mesh=mesh,
        compiler_params=pltpu.CompilerParams(use_tc_tiling_on_sc=False),
        scratch_types=[
            pltpu.VMEM((_CK,), jnp.int32),
            pltpu.VMEM((_CK, H), jnp.float32),
            pltpu.SemaphoreType.DMA,
        ],
    )
    def gk(h_hbm, src_hbm, out_hbm, idx_v, rows_v, sem):
        wid = lax.axis_index("s") * NC + lax.axis_index("c")

        def body(j, carry):
            g = j * NW + wid

            @pl.when(g < nch)
            def _():
                base = pl.multiple_of(g * _CK, _CK)
                pltpu.sync_copy(src_hbm.at[pl.ds(base, _CK)], idx_v)
                pltpu.async_copy(h_hbm.at[idx_v], rows_v, sem).wait()
                pltpu.sync_copy(rows_v, out_hbm.at[pl.ds(base, _CK), :])

            return carry

        lax.fori_loop(0, iters, body, 0)

    return gk(h, src)


def _sc_scatter(val, dst, zeros_blk, n_nodes):
    E, H = val.shape
    info = plsc.get_sparse_core_info()
    NC, NS = info.num_cores, info.num_subcores
    NW = NC * NS
    nch = E // _CK
    iters = (nch + NW - 1) // NW
    NZ = n_nodes // _ZR
    mesh = plsc.VectorSubcoreMesh(core_axis_name="c", subcore_axis_name="s")

    @functools.partial(
        pl.kernel,
        out_type=jax.ShapeDtypeStruct((NC, n_nodes, H), jnp.float32),
        mesh=mesh,
        compiler_params=pltpu.CompilerParams(use_tc_tiling_on_sc=False),
        scratch_types=[
            pltpu.VMEM((_CK,), jnp.int32),
            pltpu.VMEM((_CK, H), jnp.float32),
            pltpu.VMEM_SHARED((n_nodes, H), jnp.float32),
        ],
    )
    def sk(val_hbm, dst_hbm, z_hbm, out_hbm, idx_v, rows_v, acc):
        c = lax.axis_index("c")
        s = lax.axis_index("s")
        wid = s * NC + c

        @pl.when(s < NZ)
        def _():
            off = pl.multiple_of(s * _ZR, 8)
            pltpu.sync_copy(z_hbm, acc.at[pl.ds(off, _ZR), :])

        plsc.subcore_barrier()

        def body(j, carry):
            g = j * NW + wid

            @pl.when(g < nch)
            def _():
                base = pl.multiple_of(g * _CK, _CK)
                pltpu.sync_copy(dst_hbm.at[pl.ds(base, _CK)], idx_v)
                pltpu.sync_copy(val_hbm.at[pl.ds(base, _CK), :], rows_v)
                pltpu.sync_copy(rows_v, acc.at[idx_v], add=True)

            return carry

        lax.fori_loop(0, iters, body, 0)
        plsc.subcore_barrier()

        @pl.when(s < NZ)
        def _():
            off = pl.multiple_of(s * _ZR, 8)
            pltpu.sync_copy(acc.at[pl.ds(off, _ZR), :], out_hbm.at[c, pl.ds(off, _ZR), :])

    return sk(val, dst, zeros_blk)


def _embed(x, W_emb, b_emb, ln_g, ln_b):
    n_nodes, D = x.shape
    H = W_emb.shape[0]
    G = n_nodes // _BN

    def body(x_ref, w_ref, b_ref, g_ref, bb_ref, out_ref):
        xv = x_ref[...]
        hm = lax.dot_general(xv, w_ref[...], (((1,), (1,)), ((), ())),
                             preferred_element_type=jnp.float32) + b_ref[...]
        mu = jnp.mean(hm, axis=1, keepdims=True)
        var = jnp.mean((hm - mu) ** 2, axis=1, keepdims=True)
        hn = (hm - mu) / jnp.sqrt(var + 1e-5) * g_ref[...] + bb_ref[...]
        out_ref[...] = jnp.maximum(hn, 0.0)

    return pl.pallas_call(
        body,
        grid=(G,),
        in_specs=[
            pl.BlockSpec((_BN, D), lambda i: (i, 0)),
            pl.BlockSpec((H, D), lambda i: (0, 0)),
            pl.BlockSpec((1, H), lambda i: (0, 0)),
            pl.BlockSpec((1, H), lambda i: (0, 0)),
            pl.BlockSpec((1, H), lambda i: (0, 0)),
        ],
        out_specs=pl.BlockSpec((_BN, H), lambda i: (i, 0)),
        out_shape=jax.ShapeDtypeStruct((n_nodes, H), jnp.float32),
    )(x, W_emb, b_emb.reshape(1, H), ln_g.reshape(1, H), ln_b.reshape(1, H))


def _bspline_tr(hs, et, coeffs_ref, wb_ref, ws_ref, R):
    t0 = _KNOTS[0]
    inv_dt = 1.0 / (_KNOTS[1] - _KNOTS[0])
    base = hs * jax.nn.sigmoid(hs)
    u = (hs - t0) * inv_dt
    ifl = jnp.floor(u)
    f = u - ifl
    ii = ifl.astype(jnp.int32)
    f2 = f * f
    f3 = f2 * f
    onemf = 1.0 - f
    w0 = onemf * onemf * onemf * (1.0 / 6.0)
    w1 = 0.5 * f3 - f2 + (2.0 / 3.0)
    w2 = -0.5 * f3 + 0.5 * f2 + 0.5 * f + (1.0 / 6.0)
    w3 = f3 * (1.0 / 6.0)
    oh = [et == r for r in range(R)]
    zcol = jnp.zeros_like(hs[:, :1])
    ce = []
    for n in range(_NB):
        ce_n = zcol
        for r in range(R):
            ce_n = jnp.where(oh[r], coeffs_ref[r, n], ce_n)
        ce.append(ce_n)
    spline = jnp.zeros_like(hs)
    for k, w in enumerate((w0, w1, w2, w3)):
        j = ii + (k - 3)
        cej = jnp.zeros_like(hs)
        for n in range(_NB):
            cej = jnp.where(j == n, ce[n], cej)
        spline = spline + w * cej
    wb = zcol
    ws = zcol
    for r in range(R):
        wb = jnp.where(oh[r], wb_ref[r], wb)
        ws = jnp.where(oh[r], ws_ref[r], ws)
    return wb * base + ws * spline


def _edge_transform(hs, et3, coeffs, w_base, w_spline, attention):
    E, H = hs.shape
    G = E // _BE
    R = w_base.shape[0]

    def body(hs_ref, et_ref, coeffs_ref, wb_ref, ws_ref, att_ref,
             tr_ref, sc_ref, m_ref, s_ref, m_scr, s_scr):
        i = pl.program_id(0)

        @pl.when(i == 0)
        def _():
            for r in range(R):
                m_scr[r] = -1e30
                s_scr[r] = 0.0

        hs_v = hs_ref[...]
        et = et_ref[0]
        tr = _bspline_tr(hs_v, et, coeffs_ref, wb_ref, ws_ref, R)
        sc = jnp.sum(tr * att_ref[...], axis=1, keepdims=True)
        tr_ref[...] = tr
        sc_ref[0] = sc
        for r in range(R):
            mask = et == r
            bm = jnp.max(jnp.where(mask, sc, -1e30))
            bs = jnp.sum(jnp.where(mask, jnp.exp(sc - bm), 0.0))
            m_old = m_scr[r]
            m_new = jnp.maximum(m_old, bm)
            s_scr[r] = s_scr[r] * jnp.exp(m_old - m_new) + bs * jnp.exp(bm - m_new)
            m_scr[r] = m_new

        @pl.when(i == G - 1)
        def _():
            for r in range(R):
                m_ref[r] = m_scr[r]
                s_ref[r] = s_scr[r]

    return pl.pallas_call(
        body,
        grid=(G,),
        in_specs=[
            pl.BlockSpec((_BE, H), lambda i: (i, 0)),
            pl.BlockSpec((1, _BE, 1), lambda i: (i, 0, 0)),
            pl.BlockSpec(memory_space=pltpu.SMEM),
            pl.BlockSpec(memory_space=pltpu.SMEM),
            pl.BlockSpec(memory_space=pltpu.SMEM),
            pl.BlockSpec((1, H), lambda i: (0, 0)),
        ],
        out_specs=[
            pl.BlockSpec((_BE, H), lambda i: (i, 0)),
            pl.BlockSpec((1, _BE, 1), lambda i: (i, 0, 0)),
            pl.BlockSpec(memory_space=pltpu.SMEM),
            pl.BlockSpec(memory_space=pltpu.SMEM),
        ],
        out_shape=[
            jax.ShapeDtypeStruct((E, H), jnp.float32),
            jax.ShapeDtypeStruct((G, _BE, 1), jnp.float32),
            jax.ShapeDtypeStruct((R,), jnp.float32),
            jax.ShapeDtypeStruct((R,), jnp.float32),
        ],
        scratch_shapes=[
            pltpu.SMEM((R,), jnp.float32),
            pltpu.SMEM((R,), jnp.float32),
        ],
    )(hs, et3, coeffs, w_base, w_spline, attention.reshape(1, H))


def _scale(tr, sc3, et3, m, s):
    E, H = tr.shape
    G = E // _BE
    R = m.shape[0]

    def body(tr_ref, sc_ref, et_ref, m_ref, s_ref, val_ref, attn_ref):
        sc = sc_ref[0]
        et = et_ref[0]
        m_e = jnp.zeros_like(sc)
        s_e = jnp.ones_like(sc)
        for r in range(R):
            m_e = jnp.where(et == r, m_ref[r], m_e)
            s_e = jnp.where(et == r, s_ref[r], s_e)
        attn = jnp.exp(sc - m_e) / s_e
        val_ref[...] = tr_ref[...] * attn
        attn_ref[0] = attn

    return pl.pallas_call(
        body,
        grid=(G,),
        in_specs=[
            pl.BlockSpec((_BE, H), lambda i: (i, 0)),
            pl.BlockSpec((1, _BE, 1), lambda i: (i, 0, 0)),
            pl.BlockSpec((1, _BE, 1), lambda i: (i, 0, 0)),
            pl.BlockSpec(memory_space=pltpu.SMEM),
            pl.BlockSpec(memory_space=pltpu.SMEM),
        ],
        out_specs=[
            pl.BlockSpec((_BE, H), lambda i: (i, 0)),
            pl.BlockSpec((1, _BE, 1), lambda i: (i, 0, 0)),
        ],
        out_shape=[
            jax.ShapeDtypeStruct((E, H), jnp.float32),
            jax.ShapeDtypeStruct((G, _BE, 1), jnp.float32),
        ],
    )(tr, sc3, et3, m, s)


def _gru(msg2, h, W_ih, W_hh, b_ih, b_hh):
    n_nodes, H = h.shape
    NC = msg2.shape[0]
    G = n_nodes // _BN

    def body(msg_ref, h_ref, wih_ref, whh_ref, bih_ref, bhh_ref, out_ref):
        msg = msg_ref[0]
        for c in range(1, NC):
            msg = msg + msg_ref[c]
        hv = h_ref[...]
        gi = lax.dot_general(msg, wih_ref[...], (((1,), (1,)), ((), ())),
                             preferred_element_type=jnp.float32) + bih_ref[...]
        gh = lax.dot_general(hv, whh_ref[...], (((1,), (1,)), ((), ())),
                             preferred_element_type=jnp.float32) + bhh_ref[...]
        rg = jax.nn.sigmoid(gi[:, :H] + gh[:, :H])
        zg = jax.nn.sigmoid(gi[:, H:2 * H] + gh[:, H:2 * H])
        ng = jnp.tanh(gi[:, 2 * H:] + rg * gh[:, 2 * H:])
        out_ref[...] = (1.0 - zg) * ng + zg * hv

    return pl.pallas_call(
        body,
        grid=(G,),
        in_specs=[
            pl.BlockSpec((NC, _BN, H), lambda i: (0, i, 0)),
            pl.BlockSpec((_BN, H), lambda i: (i, 0)),
            pl.BlockSpec((3 * H, H), lambda i: (0, 0)),
            pl.BlockSpec((3 * H, H), lambda i: (0, 0)),
            pl.BlockSpec((1, 3 * H), lambda i: (0, 0)),
            pl.BlockSpec((1, 3 * H), lambda i: (0, 0)),
        ],
        out_specs=pl.BlockSpec((_BN, H), lambda i: (i, 0)),
        out_shape=jax.ShapeDtypeStruct((n_nodes, H), jnp.float32),
    )(msg2, h, W_ih, W_hh, b_ih.reshape(1, 3 * H), b_hh.reshape(1, 3 * H))


def kernel(x, edge_index, edge_type, W_emb, b_emb, ln_g, ln_b, w_base, w_spline,
           coeffs, attention, W_ih, W_hh, b_ih, b_hh):
    n_nodes, _ = x.shape
    H = W_emb.shape[0]
    E = edge_type.shape[0]
    R = w_base.shape[0]
    src = edge_index[0].astype(jnp.int32)
    dst = edge_index[1].astype(jnp.int32)
    et = edge_type.astype(jnp.int32)
    et3 = et.reshape(E // _BE, _BE, 1)
    zeros_blk = jnp.zeros((_ZR, H), jnp.float32)

    h = _embed(x, W_emb, b_emb, ln_g, ln_b)
    attns = []
    for _ in range(2):
        hs = _sc_gather(h, src)
        tr, sc3, m, s = _edge_transform(hs, et3, coeffs, w_base, w_spline, attention)
        val, attn3 = _scale(tr, sc3, et3, m, s)
        msg2 = _sc_scatter(val, dst, zeros_blk, n_nodes)
        h = _gru(msg2, h, W_ih, W_hh, b_ih, b_hh)
        attns.append(attn3.reshape(E))
    return h, jnp.stack(attns)


# speedup vs baseline: 3.2606x; 1.6108x over previous
import functools

import numpy as np
import jax
import jax.numpy as jnp
from jax import lax
from jax.experimental import pallas as pl
from jax.experimental.pallas import tpu as pltpu
from jax.experimental.pallas import tpu_sc as plsc

_DEG = 3
_NB = 7
_KNOTS = [float(v) for v in np.linspace(-7.0, 7.0, _NB + _DEG + 1).astype(np.float32)]

_BE = 1000
_BN = 1000
_CK = 128
_ZR = 1000


def _sc_gather(h, src):
    n_nodes, H = h.shape
    E = src.shape[0]
    info = plsc.get_sparse_core_info()
    NC, NS = info.num_cores, info.num_subcores
    NW = NC * NS
    nch = E // _CK
    iters = (nch + NW - 1) // NW
    mesh = plsc.VectorSubcoreMesh(core_axis_name="c", subcore_axis_name="s")

    @functools.partial(
        pl.kernel,
        out_type=jax.ShapeDtypeStruct((E, H), jnp.float32),
        mesh=mesh,
        compiler_params=pltpu.CompilerParams(use_tc_tiling_on_sc=False),
        scratch_types=[
            pltpu.VMEM((_CK,), jnp.int32),
            pltpu.VMEM((_CK, H), jnp.float32),
            pltpu.SemaphoreType.DMA,
        ],
    )
    def gk(h_hbm, src_hbm, out_hbm, idx_v, rows_v, sem):
        wid = lax.axis_index("s") * NC + lax.axis_index("c")

        def body(j, carry):
            g = j * NW + wid

            @pl.when(g < nch)
            def _():
                base = pl.multiple_of(g * _CK, _CK)
                pltpu.sync_copy(src_hbm.at[pl.ds(base, _CK)], idx_v)
                pltpu.async_copy(h_hbm.at[idx_v], rows_v, sem).wait()
                pltpu.sync_copy(rows_v, out_hbm.at[pl.ds(base, _CK), :])

            return carry

        lax.fori_loop(0, iters, body, 0)

    return gk(h, src)


def _sc_scatter(val, dst, zeros_blk, n_nodes):
    E, H = val.shape
    info = plsc.get_sparse_core_info()
    NC, NS = info.num_cores, info.num_subcores
    NW = NC * NS
    nch = E // _CK
    iters = (nch + NW - 1) // NW
    NZ = n_nodes // _ZR
    mesh = plsc.VectorSubcoreMesh(core_axis_name="c", subcore_axis_name="s")

    @functools.partial(
        pl.kernel,
        out_type=jax.ShapeDtypeStruct((NC, n_nodes, H), jnp.float32),
        mesh=mesh,
        compiler_params=pltpu.CompilerParams(use_tc_tiling_on_sc=False),
        scratch_types=[
            pltpu.VMEM((_CK,), jnp.int32),
            pltpu.VMEM((_CK, H), jnp.float32),
            pltpu.VMEM_SHARED((n_nodes, H), jnp.float32),
        ],
    )
    def sk(val_hbm, dst_hbm, z_hbm, out_hbm, idx_v, rows_v, acc):
        c = lax.axis_index("c")
        s = lax.axis_index("s")
        wid = s * NC + c

        @pl.when(s < NZ)
        def _():
            off = pl.multiple_of(s * _ZR, 8)
            pltpu.sync_copy(z_hbm, acc.at[pl.ds(off, _ZR), :])

        plsc.subcore_barrier()

        def body(j, carry):
            g = j * NW + wid

            @pl.when(g < nch)
            def _():
                base = pl.multiple_of(g * _CK, _CK)
                pltpu.sync_copy(dst_hbm.at[pl.ds(base, _CK)], idx_v)
                pltpu.sync_copy(val_hbm.at[pl.ds(base, _CK), :], rows_v)
                pltpu.sync_copy(rows_v, acc.at[idx_v], add=True)

            return carry

        lax.fori_loop(0, iters, body, 0)
        plsc.subcore_barrier()

        @pl.when(s < NZ)
        def _():
            off = pl.multiple_of(s * _ZR, 8)
            pltpu.sync_copy(acc.at[pl.ds(off, _ZR), :], out_hbm.at[c, pl.ds(off, _ZR), :])

    return sk(val, dst, zeros_blk)


def _embed(x, W_emb, b_emb, ln_g, ln_b):
    n_nodes, D = x.shape
    H = W_emb.shape[0]
    G = n_nodes // _BN

    def body(x_ref, w_ref, b_ref, g_ref, bb_ref, out_ref):
        xv = x_ref[...]
        hm = lax.dot_general(xv, w_ref[...], (((1,), (1,)), ((), ())),
                             preferred_element_type=jnp.float32) + b_ref[...]
        mu = jnp.mean(hm, axis=1, keepdims=True)
        var = jnp.mean((hm - mu) ** 2, axis=1, keepdims=True)
        hn = (hm - mu) / jnp.sqrt(var + 1e-5) * g_ref[...] + bb_ref[...]
        out_ref[...] = jnp.maximum(hn, 0.0)

    return pl.pallas_call(
        body,
        grid=(G,),
        in_specs=[
            pl.BlockSpec((_BN, D), lambda i: (i, 0)),
            pl.BlockSpec((H, D), lambda i: (0, 0)),
            pl.BlockSpec((1, H), lambda i: (0, 0)),
            pl.BlockSpec((1, H), lambda i: (0, 0)),
            pl.BlockSpec((1, H), lambda i: (0, 0)),
        ],
        out_specs=pl.BlockSpec((_BN, H), lambda i: (i, 0)),
        out_shape=jax.ShapeDtypeStruct((n_nodes, H), jnp.float32),
    )(x, W_emb, b_emb.reshape(1, H), ln_g.reshape(1, H), ln_b.reshape(1, H))


def _bspline_tr(hs, left, et_e, et_o, coeffs_ref, wb_ref, ws_ref, R):
    t0 = _KNOTS[0]
    inv_dt = 1.0 / (_KNOTS[1] - _KNOTS[0])
    base = hs * jax.nn.sigmoid(hs)
    u = (hs - t0) * inv_dt
    ifl = jnp.floor(u)
    f = u - ifl
    ii = ifl.astype(jnp.int32)
    f2 = f * f
    f3 = f2 * f
    onemf = 1.0 - f
    w0 = onemf * onemf * onemf * (1.0 / 6.0)
    w1 = 0.5 * f3 - f2 + (2.0 / 3.0)
    w2 = -0.5 * f3 + 0.5 * f2 + 0.5 * f + (1.0 / 6.0)
    w3 = f3 * (1.0 / 6.0)
    oh_e = [et_e == r for r in range(R)]
    oh_o = [et_o == r for r in range(R)]
    zcol = jnp.zeros_like(et_e, dtype=hs.dtype)

    def _sel(vals_by_r):
        col_e = zcol
        col_o = zcol
        for r in range(R):
            col_e = jnp.where(oh_e[r], vals_by_r[r], col_e)
            col_o = jnp.where(oh_o[r], vals_by_r[r], col_o)
        return jnp.where(left, col_e, col_o)

    ce = [_sel([coeffs_ref[r, n] for r in range(R)]) for n in range(_NB)]
    spline = jnp.zeros_like(hs)
    for k, w in enumerate((w0, w1, w2, w3)):
        j = ii + (k - 3)
        cej = jnp.zeros_like(hs)
        for n in range(_NB):
            cej = jnp.where(j == n, ce[n], cej)
        spline = spline + w * cej
    wb = _sel([wb_ref[r] for r in range(R)])
    ws = _sel([ws_ref[r] for r in range(R)])
    return wb * base + ws * spline


def _edge_transform(hs2, etp, coeffs, w_base, w_spline, attention):
    EP, W = hs2.shape
    H = W // 2
    G = EP // _BE
    R = w_base.shape[0]

    def body(hs_ref, et_ref, coeffs_ref, wb_ref, ws_ref, att_ref,
             tr_ref, sc_ref, m_ref, s_ref, m_scr, s_scr):
        i = pl.program_id(0)

        @pl.when(i == 0)
        def _():
            for r in range(R):
                m_scr[r] = -1e30
                s_scr[r] = 0.0

        hs_v = hs_ref[...]
        etpair = et_ref[0]
        et_e = etpair[:, 0:1]
        et_o = etpair[:, 1:2]
        lane = lax.broadcasted_iota(jnp.int32, (_BE, W), 1)
        left = lane < H
        tr = _bspline_tr(hs_v, left, et_e, et_o, coeffs_ref, wb_ref, ws_ref, R)
        trw = tr * att_ref[...]
        sc_e = jnp.sum(trw[:, :H], axis=1, keepdims=True)
        sc_o = jnp.sum(trw[:, H:], axis=1, keepdims=True)
        sc = jnp.concatenate([sc_e, sc_o], axis=1)
        tr_ref[...] = tr
        sc_ref[0] = sc
        for r in range(R):
            mask = etpair == r
            bm = jnp.max(jnp.where(mask, sc, -1e30))
            bs = jnp.sum(jnp.where(mask, jnp.exp(sc - bm), 0.0))
            m_old = m_scr[r]
            m_new = jnp.maximum(m_old, bm)
            s_scr[r] = s_scr[r] * jnp.exp(m_old - m_new) + bs * jnp.exp(bm - m_new)
            m_scr[r] = m_new

        @pl.when(i == G - 1)
        def _():
            for r in range(R):
                m_ref[r] = m_scr[r]
                s_ref[r] = s_scr[r]

    att2 = jnp.concatenate([attention, attention]).reshape(1, W)
    return pl.pallas_call(
        body,
        grid=(G,),
        in_specs=[
            pl.BlockSpec((_BE, W), lambda i: (i, 0)),
            pl.BlockSpec((1, _BE, 2), lambda i: (i, 0, 0)),
            pl.BlockSpec(memory_space=pltpu.SMEM),
            pl.BlockSpec(memory_space=pltpu.SMEM),
            pl.BlockSpec(memory_space=pltpu.SMEM),
            pl.BlockSpec((1, W), lambda i: (0, 0)),
        ],
        out_specs=[
            pl.BlockSpec((_BE, W), lambda i: (i, 0)),
            pl.BlockSpec((1, _BE, 2), lambda i: (i, 0, 0)),
            pl.BlockSpec(memory_space=pltpu.SMEM),
            pl.BlockSpec(memory_space=pltpu.SMEM),
        ],
        out_shape=[
            jax.ShapeDtypeStruct((EP, W), jnp.float32),
            jax.ShapeDtypeStruct((G, _BE, 2), jnp.float32),
            jax.ShapeDtypeStruct((R,), jnp.float32),
            jax.ShapeDtypeStruct((R,), jnp.float32),
        ],
        scratch_shapes=[
            pltpu.SMEM((R,), jnp.float32),
            pltpu.SMEM((R,), jnp.float32),
        ],
    )(hs2, etp, coeffs, w_base, w_spline, att2)


def _scale(tr2, sc3, etp, m, s):
    EP, W = tr2.shape
    H = W // 2
    G = EP // _BE
    R = m.shape[0]

    def body(tr_ref, sc_ref, et_ref, m_ref, s_ref, val_ref, attn_ref):
        sc = sc_ref[0]
        et = et_ref[0]
        m_e = jnp.zeros_like(sc)
        s_e = jnp.ones_like(sc)
        for r in range(R):
            m_e = jnp.where(et == r, m_ref[r], m_e)
            s_e = jnp.where(et == r, s_ref[r], s_e)
        attn = jnp.exp(sc - m_e) / s_e
        lane = lax.broadcasted_iota(jnp.int32, (_BE, W), 1)
        attn_wide = jnp.where(lane < H, attn[:, 0:1], attn[:, 1:2])
        val_ref[...] = tr_ref[...] * attn_wide
        attn_ref[0] = attn

    return pl.pallas_call(
        body,
        grid=(G,),
        in_specs=[
            pl.BlockSpec((_BE, W), lambda i: (i, 0)),
            pl.BlockSpec((1, _BE, 2), lambda i: (i, 0, 0)),
            pl.BlockSpec((1, _BE, 2), lambda i: (i, 0, 0)),
            pl.BlockSpec(memory_space=pltpu.SMEM),
            pl.BlockSpec(memory_space=pltpu.SMEM),
        ],
        out_specs=[
            pl.BlockSpec((_BE, W), lambda i: (i, 0)),
            pl.BlockSpec((1, _BE, 2), lambda i: (i, 0, 0)),
        ],
        out_shape=[
            jax.ShapeDtypeStruct((EP, W), jnp.float32),
            jax.ShapeDtypeStruct((G, _BE, 2), jnp.float32),
        ],
    )(tr2, sc3, etp, m, s)


def _gru(msg2, h, W_ih, W_hh, b_ih, b_hh):
    n_nodes, H = h.shape
    NC = msg2.shape[0]
    G = n_nodes // _BN

    def body(msg_ref, h_ref, wih_ref, whh_ref, bih_ref, bhh_ref, out_ref):
        msg = msg_ref[0]
        for c in range(1, NC):
            msg = msg + msg_ref[c]
        hv = h_ref[...]
        gi = lax.dot_general(msg, wih_ref[...], (((1,), (1,)), ((), ())),
                             preferred_element_type=jnp.float32) + bih_ref[...]
        gh = lax.dot_general(hv, whh_ref[...], (((1,), (1,)), ((), ())),
                             preferred_element_type=jnp.float32) + bhh_ref[...]
        rg = jax.nn.sigmoid(gi[:, :H] + gh[:, :H])
        zg = jax.nn.sigmoid(gi[:, H:2 * H] + gh[:, H:2 * H])
        ng = jnp.tanh(gi[:, 2 * H:] + rg * gh[:, 2 * H:])
        out_ref[...] = (1.0 - zg) * ng + zg * hv

    return pl.pallas_call(
        body,
        grid=(G,),
        in_specs=[
            pl.BlockSpec((NC, _BN, H), lambda i: (0, i, 0)),
            pl.BlockSpec((_BN, H), lambda i: (i, 0)),
            pl.BlockSpec((3 * H, H), lambda i: (0, 0)),
            pl.BlockSpec((3 * H, H), lambda i: (0, 0)),
            pl.BlockSpec((1, 3 * H), lambda i: (0, 0)),
            pl.BlockSpec((1, 3 * H), lambda i: (0, 0)),
        ],
        out_specs=pl.BlockSpec((_BN, H), lambda i: (i, 0)),
        out_shape=jax.ShapeDtypeStruct((n_nodes, H), jnp.float32),
    )(msg2, h, W_ih, W_hh, b_ih.reshape(1, 3 * H), b_hh.reshape(1, 3 * H))


def kernel(x, edge_index, edge_type, W_emb, b_emb, ln_g, ln_b, w_base, w_spline,
           coeffs, attention, W_ih, W_hh, b_ih, b_hh):
    n_nodes, _ = x.shape
    H = W_emb.shape[0]
    E = edge_type.shape[0]
    R = w_base.shape[0]
    src = edge_index[0].astype(jnp.int32)
    dst = edge_index[1].astype(jnp.int32)
    et = edge_type.astype(jnp.int32)
    etp = et.reshape(E // (2 * _BE), _BE, 2)
    zeros_blk = jnp.zeros((_ZR, H), jnp.float32)

    h = _embed(x, W_emb, b_emb, ln_g, ln_b)
    attns = []
    for _ in range(2):
        hs = _sc_gather(h, src)
        hs2 = hs.reshape(E // 2, 2 * H)
        tr2, sc3, m, s = _edge_transform(hs2, etp, coeffs, w_base, w_spline, attention)
        val2, attn3 = _scale(tr2, sc3, etp, m, s)
        msg2 = _sc_scatter(val2.reshape(E, H), dst, zeros_blk, n_nodes)
        h = _gru(msg2, h, W_ih, W_hh, b_ih, b_hh)
        attns.append(attn3.reshape(E))
    return h, jnp.stack(attns)


# speedup vs baseline: 3.4392x; 1.0548x over previous
import functools

import numpy as np
import jax
import jax.numpy as jnp
from jax import lax
from jax.experimental import pallas as pl
from jax.experimental.pallas import tpu as pltpu
from jax.experimental.pallas import tpu_sc as plsc

_DEG = 3
_NB = 7
_KNOTS = [float(v) for v in np.linspace(-7.0, 7.0, _NB + _DEG + 1).astype(np.float32)]

_BE = 1000
_BN = 1000
_CK = 128
_ZR = 1000


def _sc_gather(h, src):
    n_nodes, H = h.shape
    E = src.shape[0]
    info = plsc.get_sparse_core_info()
    NC, NS = info.num_cores, info.num_subcores
    NW = NC * NS
    nch = E // _CK
    iters = (nch + NW - 1) // NW
    mesh = plsc.VectorSubcoreMesh(core_axis_name="c", subcore_axis_name="s")

    @functools.partial(
        pl.kernel,
        out_type=jax.ShapeDtypeStruct((E, H), jnp.float32),
        mesh=mesh,
        compiler_params=pltpu.CompilerParams(use_tc_tiling_on_sc=False),
        scratch_types=[
            pltpu.VMEM((2, _CK), jnp.int32),
            pltpu.VMEM((_CK, H), jnp.float32),
            pltpu.VMEM((_CK, H), jnp.float32),
            pltpu.SemaphoreType.DMA,
            pltpu.SemaphoreType.DMA,
            pltpu.SemaphoreType.DMA,
            pltpu.SemaphoreType.DMA,
        ],
    )
    def gk(h_hbm, src_hbm, out_hbm, idx_v, rows_a, rows_b, sga, sgb, swa, swb):
        wid = lax.axis_index("s") * NC + lax.axis_index("c")
        rows = (rows_a, rows_b)
        sg = (sga, sgb)
        sw = (swa, swb)

        def body(j, carry):
            cs = (2 * j) * NW + wid
            for b in range(2):
                c = cs + b * NW

                @pl.when(c < nch)
                def _(c=c, b=b):
                    base = pl.multiple_of(c * _CK, _CK)
                    pltpu.sync_copy(src_hbm.at[pl.ds(base, _CK)], idx_v.at[b])

                    @pl.when(j > 0)
                    def _():
                        pltpu.make_async_copy(rows[b], out_hbm.at[pl.ds(base, _CK), :], sw[b]).wait()

                    pltpu.async_copy(h_hbm.at[idx_v.at[b]], rows[b], sg[b])

            for b in range(2):
                c = cs + b * NW

                @pl.when(c < nch)
                def _(c=c, b=b):
                    base = pl.multiple_of(c * _CK, _CK)
                    pltpu.make_async_copy(h_hbm.at[idx_v.at[b]], rows[b], sg[b]).wait()
                    pltpu.async_copy(rows[b], out_hbm.at[pl.ds(base, _CK), :], sw[b])

            return carry

        lax.fori_loop(0, (iters + 1) // 2, body, 0)
        for b in range(2):
            pltpu.make_async_copy(rows[b], out_hbm.at[pl.ds(0, _CK), :], sw[b]).wait()

    return gk(h, src)


def _sc_scatter(val, dst, zeros_blk, n_nodes):
    E, H = val.shape
    info = plsc.get_sparse_core_info()
    NC, NS = info.num_cores, info.num_subcores
    NW = NC * NS
    nch = E // _CK
    iters = (nch + NW - 1) // NW
    NZ = n_nodes // _ZR
    mesh = plsc.VectorSubcoreMesh(core_axis_name="c", subcore_axis_name="s")

    @functools.partial(
        pl.kernel,
        out_type=jax.ShapeDtypeStruct((NC, n_nodes, H), jnp.float32),
        mesh=mesh,
        compiler_params=pltpu.CompilerParams(use_tc_tiling_on_sc=False),
        scratch_types=[
            pltpu.VMEM((2, _CK), jnp.int32),
            pltpu.VMEM((_CK, H), jnp.float32),
            pltpu.VMEM((_CK, H), jnp.float32),
            pltpu.SemaphoreType.DMA,
            pltpu.SemaphoreType.DMA,
            pltpu.VMEM_SHARED((n_nodes, H), jnp.float32),
        ],
    )
    def sk(val_hbm, dst_hbm, z_hbm, out_hbm, idx_v, rows_a, rows_b, sva, svb, acc):
        c = lax.axis_index("c")
        s = lax.axis_index("s")
        wid = s * NC + c
        rows = (rows_a, rows_b)
        sv = (sva, svb)

        @pl.when(s < NZ)
        def _():
            off = pl.multiple_of(s * _ZR, 8)
            pltpu.sync_copy(z_hbm, acc.at[pl.ds(off, _ZR), :])

        plsc.subcore_barrier()

        def body(j, carry):
            cs = (2 * j) * NW + wid
            for b in range(2):
                ck = cs + b * NW

                @pl.when(ck < nch)
                def _(ck=ck, b=b):
                    base = pl.multiple_of(ck * _CK, _CK)
                    pltpu.sync_copy(dst_hbm.at[pl.ds(base, _CK)], idx_v.at[b])
                    pltpu.async_copy(val_hbm.at[pl.ds(base, _CK), :], rows[b], sv[b])

            for b in range(2):
                ck = cs + b * NW

                @pl.when(ck < nch)
                def _(ck=ck, b=b):
                    base = pl.multiple_of(ck * _CK, _CK)
                    pltpu.make_async_copy(val_hbm.at[pl.ds(base, _CK), :], rows[b], sv[b]).wait()
                    pltpu.sync_copy(rows[b], acc.at[idx_v.at[b]], add=True)

            return carry

        lax.fori_loop(0, (iters + 1) // 2, body, 0)
        plsc.subcore_barrier()

        @pl.when(s < NZ)
        def _():
            off = pl.multiple_of(s * _ZR, 8)
            pltpu.sync_copy(acc.at[pl.ds(off, _ZR), :], out_hbm.at[c, pl.ds(off, _ZR), :])

    return sk(val, dst, zeros_blk)


def _embed(x, W_emb, b_emb, ln_g, ln_b):
    n_nodes, D = x.shape
    H = W_emb.shape[0]
    G = n_nodes // _BN

    def body(x_ref, w_ref, b_ref, g_ref, bb_ref, out_ref):
        xv = x_ref[...]
        hm = lax.dot_general(xv, w_ref[...], (((1,), (1,)), ((), ())),
                             preferred_element_type=jnp.float32) + b_ref[...]
        mu = jnp.mean(hm, axis=1, keepdims=True)
        var = jnp.mean((hm - mu) ** 2, axis=1, keepdims=True)
        hn = (hm - mu) / jnp.sqrt(var + 1e-5) * g_ref[...] + bb_ref[...]
        out_ref[...] = jnp.maximum(hn, 0.0)

    return pl.pallas_call(
        body,
        grid=(G,),
        in_specs=[
            pl.BlockSpec((_BN, D), lambda i: (i, 0)),
            pl.BlockSpec((H, D), lambda i: (0, 0)),
            pl.BlockSpec((1, H), lambda i: (0, 0)),
            pl.BlockSpec((1, H), lambda i: (0, 0)),
            pl.BlockSpec((1, H), lambda i: (0, 0)),
        ],
        out_specs=pl.BlockSpec((_BN, H), lambda i: (i, 0)),
        out_shape=jax.ShapeDtypeStruct((n_nodes, H), jnp.float32),
    )(x, W_emb, b_emb.reshape(1, H), ln_g.reshape(1, H), ln_b.reshape(1, H))


def _bspline_tr(hs, left, et_e, et_o, coeffs_ref, wb_ref, ws_ref, R):
    t0 = _KNOTS[0]
    inv_dt = 1.0 / (_KNOTS[1] - _KNOTS[0])
    base = hs * jax.nn.sigmoid(hs)
    u = (hs - t0) * inv_dt
    ifl = jnp.floor(u)
    f = u - ifl
    ii = ifl.astype(jnp.int32)
    f2 = f * f
    f3 = f2 * f
    onemf = 1.0 - f
    w0 = onemf * onemf * onemf * (1.0 / 6.0)
    w1 = 0.5 * f3 - f2 + (2.0 / 3.0)
    w2 = -0.5 * f3 + 0.5 * f2 + 0.5 * f + (1.0 / 6.0)
    w3 = f3 * (1.0 / 6.0)
    oh_e = [et_e == r for r in range(R)]
    oh_o = [et_o == r for r in range(R)]
    zcol = jnp.zeros_like(et_e, dtype=hs.dtype)

    def _sel(vals_by_r):
        col_e = zcol
        col_o = zcol
        for r in range(R):
            col_e = jnp.where(oh_e[r], vals_by_r[r], col_e)
            col_o = jnp.where(oh_o[r], vals_by_r[r], col_o)
        return jnp.where(left, col_e, col_o)

    ce = [_sel([coeffs_ref[r, n] for r in range(R)]) for n in range(_NB)]
    spline = jnp.zeros_like(hs)
    for k, w in enumerate((w0, w1, w2, w3)):
        j = ii + (k - 3)
        cej = jnp.zeros_like(hs)
        for n in range(_NB):
            cej = jnp.where(j == n, ce[n], cej)
        spline = spline + w * cej
    wb = _sel([wb_ref[r] for r in range(R)])
    ws = _sel([ws_ref[r] for r in range(R)])
    return wb * base + ws * spline


def _edge_transform(hs2, etp, coeffs, w_base, w_spline, attention):
    EP, W = hs2.shape
    H = W // 2
    G = EP // _BE
    R = w_base.shape[0]

    def body(hs_ref, et_ref, coeffs_ref, wb_ref, ws_ref, att_ref,
             tr_ref, sc_ref, m_ref, s_ref, m_scr, s_scr):
        i = pl.program_id(0)

        @pl.when(i == 0)
        def _():
            for r in range(R):
                m_scr[r] = -1e30
                s_scr[r] = 0.0

        hs_v = hs_ref[...]
        etpair = et_ref[0]
        et_e = etpair[:, 0:1]
        et_o = etpair[:, 1:2]
        lane = lax.broadcasted_iota(jnp.int32, (_BE, W), 1)
        left = lane < H
        tr = _bspline_tr(hs_v, left, et_e, et_o, coeffs_ref, wb_ref, ws_ref, R)
        trw = tr * att_ref[...]
        sc_e = jnp.sum(trw[:, :H], axis=1, keepdims=True)
        sc_o = jnp.sum(trw[:, H:], axis=1, keepdims=True)
        sc = jnp.concatenate([sc_e, sc_o], axis=1)
        tr_ref[...] = tr
        sc_ref[0] = sc
        for r in range(R):
            mask = etpair == r
            bm = jnp.max(jnp.where(mask, sc, -1e30))
            bs = jnp.sum(jnp.where(mask, jnp.exp(sc - bm), 0.0))
            m_old = m_scr[r]
            m_new = jnp.maximum(m_old, bm)
            s_scr[r] = s_scr[r] * jnp.exp(m_old - m_new) + bs * jnp.exp(bm - m_new)
            m_scr[r] = m_new

        @pl.when(i == G - 1)
        def _():
            for r in range(R):
                m_ref[r] = m_scr[r]
                s_ref[r] = s_scr[r]

    att2 = jnp.concatenate([attention, attention]).reshape(1, W)
    return pl.pallas_call(
        body,
        grid=(G,),
        in_specs=[
            pl.BlockSpec((_BE, W), lambda i: (i, 0)),
            pl.BlockSpec((1, _BE, 2), lambda i: (i, 0, 0)),
            pl.BlockSpec(memory_space=pltpu.SMEM),
            pl.BlockSpec(memory_space=pltpu.SMEM),
            pl.BlockSpec(memory_space=pltpu.SMEM),
            pl.BlockSpec((1, W), lambda i: (0, 0)),
        ],
        out_specs=[
            pl.BlockSpec((_BE, W), lambda i: (i, 0)),
            pl.BlockSpec((1, _BE, 2), lambda i: (i, 0, 0)),
            pl.BlockSpec(memory_space=pltpu.SMEM),
            pl.BlockSpec(memory_space=pltpu.SMEM),
        ],
        out_shape=[
            jax.ShapeDtypeStruct((EP, W), jnp.float32),
            jax.ShapeDtypeStruct((G, _BE, 2), jnp.float32),
            jax.ShapeDtypeStruct((R,), jnp.float32),
            jax.ShapeDtypeStruct((R,), jnp.float32),
        ],
        scratch_shapes=[
            pltpu.SMEM((R,), jnp.float32),
            pltpu.SMEM((R,), jnp.float32),
        ],
    )(hs2, etp, coeffs, w_base, w_spline, att2)


def _scale(tr2, sc3, etp, m, s):
    EP, W = tr2.shape
    H = W // 2
    G = EP // _BE
    R = m.shape[0]

    def body(tr_ref, sc_ref, et_ref, m_ref, s_ref, val_ref, attn_ref):
        sc = sc_ref[0]
        et = et_ref[0]
        m_e = jnp.zeros_like(sc)
        s_e = jnp.ones_like(sc)
        for r in range(R):
            m_e = jnp.where(et == r, m_ref[r], m_e)
            s_e = jnp.where(et == r, s_ref[r], s_e)
        attn = jnp.exp(sc - m_e) / s_e
        lane = lax.broadcasted_iota(jnp.int32, (_BE, W), 1)
        attn_wide = jnp.where(lane < H, attn[:, 0:1], attn[:, 1:2])
        val_ref[...] = tr_ref[...] * attn_wide
        attn_ref[0] = attn

    return pl.pallas_call(
        body,
        grid=(G,),
        in_specs=[
            pl.BlockSpec((_BE, W), lambda i: (i, 0)),
            pl.BlockSpec((1, _BE, 2), lambda i: (i, 0, 0)),
            pl.BlockSpec((1, _BE, 2), lambda i: (i, 0, 0)),
            pl.BlockSpec(memory_space=pltpu.SMEM),
            pl.BlockSpec(memory_space=pltpu.SMEM),
        ],
        out_specs=[
            pl.BlockSpec((_BE, W), lambda i: (i, 0)),
            pl.BlockSpec((1, _BE, 2), lambda i: (i, 0, 0)),
        ],
        out_shape=[
            jax.ShapeDtypeStruct((EP, W), jnp.float32),
            jax.ShapeDtypeStruct((G, _BE, 2), jnp.float32),
        ],
    )(tr2, sc3, etp, m, s)


def _gru(msg2, h, W_ih, W_hh, b_ih, b_hh):
    n_nodes, H = h.shape
    NC = msg2.shape[0]
    G = n_nodes // _BN

    def body(msg_ref, h_ref, wih_ref, whh_ref, bih_ref, bhh_ref, out_ref):
        msg = msg_ref[0]
        for c in range(1, NC):
            msg = msg + msg_ref[c]
        hv = h_ref[...]
        gi = lax.dot_general(msg, wih_ref[...], (((1,), (1,)), ((), ())),
                             preferred_element_type=jnp.float32) + bih_ref[...]
        gh = lax.dot_general(hv, whh_ref[...], (((1,), (1,)), ((), ())),
                             preferred_element_type=jnp.float32) + bhh_ref[...]
        rg = jax.nn.sigmoid(gi[:, :H] + gh[:, :H])
        zg = jax.nn.sigmoid(gi[:, H:2 * H] + gh[:, H:2 * H])
        ng = jnp.tanh(gi[:, 2 * H:] + rg * gh[:, 2 * H:])
        out_ref[...] = (1.0 - zg) * ng + zg * hv

    return pl.pallas_call(
        body,
        grid=(G,),
        in_specs=[
            pl.BlockSpec((NC, _BN, H), lambda i: (0, i, 0)),
            pl.BlockSpec((_BN, H), lambda i: (i, 0)),
            pl.BlockSpec((3 * H, H), lambda i: (0, 0)),
            pl.BlockSpec((3 * H, H), lambda i: (0, 0)),
            pl.BlockSpec((1, 3 * H), lambda i: (0, 0)),
            pl.BlockSpec((1, 3 * H), lambda i: (0, 0)),
        ],
        out_specs=pl.BlockSpec((_BN, H), lambda i: (i, 0)),
        out_shape=jax.ShapeDtypeStruct((n_nodes, H), jnp.float32),
    )(msg2, h, W_ih, W_hh, b_ih.reshape(1, 3 * H), b_hh.reshape(1, 3 * H))


def kernel(x, edge_index, edge_type, W_emb, b_emb, ln_g, ln_b, w_base, w_spline,
           coeffs, attention, W_ih, W_hh, b_ih, b_hh):
    n_nodes, _ = x.shape
    H = W_emb.shape[0]
    E = edge_type.shape[0]
    R = w_base.shape[0]
    src = edge_index[0].astype(jnp.int32)
    dst = edge_index[1].astype(jnp.int32)
    et = edge_type.astype(jnp.int32)
    etp = et.reshape(E // (2 * _BE), _BE, 2)
    zeros_blk = jnp.zeros((_ZR, H), jnp.float32)

    h = _embed(x, W_emb, b_emb, ln_g, ln_b)
    attns = []
    for _ in range(2):
        hs = _sc_gather(h, src)
        hs2 = hs.reshape(E // 2, 2 * H)
        tr2, sc3, m, s = _edge_transform(hs2, etp, coeffs, w_base, w_spline, attention)
        val2, attn3 = _scale(tr2, sc3, etp, m, s)
        msg2 = _sc_scatter(val2.reshape(E, H), dst, zeros_blk, n_nodes)
        h = _gru(msg2, h, W_ih, W_hh, b_ih, b_hh)
        attns.append(attn3.reshape(E))
    return h, jnp.stack(attns)


# speedup vs baseline: 4.5643x; 1.3271x over previous
import functools

import numpy as np
import jax
import jax.numpy as jnp
from jax import lax
from jax.experimental import pallas as pl
from jax.experimental.pallas import tpu as pltpu
from jax.experimental.pallas import tpu_sc as plsc

_DEG = 3
_NB = 7
_KNOTS = [float(v) for v in np.linspace(-7.0, 7.0, _NB + _DEG + 1).astype(np.float32)]

_BE = 1000
_BN = 1000
_CK = 128
_ZR = 1000


def _sc_gather(h, src):
    n_nodes, H = h.shape
    E = src.shape[0]
    info = plsc.get_sparse_core_info()
    NC, NS = info.num_cores, info.num_subcores
    NW = NC * NS
    nch = E // _CK
    iters = (nch + NW - 1) // NW
    mesh = plsc.VectorSubcoreMesh(core_axis_name="c", subcore_axis_name="s")

    @functools.partial(
        pl.kernel,
        out_type=jax.ShapeDtypeStruct((E, H), jnp.float32),
        mesh=mesh,
        compiler_params=pltpu.CompilerParams(use_tc_tiling_on_sc=False),
        scratch_types=[
            pltpu.VMEM((2, _CK), jnp.int32),
            pltpu.VMEM((_CK, H), jnp.float32),
            pltpu.VMEM((_CK, H), jnp.float32),
            pltpu.SemaphoreType.DMA,
            pltpu.SemaphoreType.DMA,
            pltpu.SemaphoreType.DMA,
            pltpu.SemaphoreType.DMA,
        ],
    )
    def gk(h_hbm, src_hbm, out_hbm, idx_v, rows_a, rows_b, sga, sgb, swa, swb):
        wid = lax.axis_index("s") * NC + lax.axis_index("c")
        rows = (rows_a, rows_b)
        sg = (sga, sgb)
        sw = (swa, swb)

        def body(j, carry):
            cs = (2 * j) * NW + wid
            for b in range(2):
                c = cs + b * NW

                @pl.when(c < nch)
                def _(c=c, b=b):
                    base = pl.multiple_of(c * _CK, _CK)
                    pltpu.sync_copy(src_hbm.at[pl.ds(base, _CK)], idx_v.at[b])

                    @pl.when(j > 0)
                    def _():
                        pltpu.make_async_copy(rows[b], out_hbm.at[pl.ds(base, _CK), :], sw[b]).wait()

                    pltpu.async_copy(h_hbm.at[idx_v.at[b]], rows[b], sg[b])

            for b in range(2):
                c = cs + b * NW

                @pl.when(c < nch)
                def _(c=c, b=b):
                    base = pl.multiple_of(c * _CK, _CK)
                    pltpu.make_async_copy(h_hbm.at[idx_v.at[b]], rows[b], sg[b]).wait()
                    pltpu.async_copy(rows[b], out_hbm.at[pl.ds(base, _CK), :], sw[b])

            return carry

        lax.fori_loop(0, (iters + 1) // 2, body, 0)
        for b in range(2):
            pltpu.make_async_copy(rows[b], out_hbm.at[pl.ds(0, _CK), :], sw[b]).wait()

    return gk(h, src)


def _sc_scatter(val, dst, zeros_blk, n_nodes):
    E, H = val.shape
    info = plsc.get_sparse_core_info()
    NC, NS = info.num_cores, info.num_subcores
    NW = NC * NS
    nch = E // _CK
    iters = (nch + NW - 1) // NW
    NZ = n_nodes // _ZR
    mesh = plsc.VectorSubcoreMesh(core_axis_name="c", subcore_axis_name="s")

    @functools.partial(
        pl.kernel,
        out_type=jax.ShapeDtypeStruct((NC, n_nodes, H), jnp.float32),
        mesh=mesh,
        compiler_params=pltpu.CompilerParams(use_tc_tiling_on_sc=False),
        scratch_types=[
            pltpu.VMEM((2, _CK), jnp.int32),
            pltpu.VMEM((_CK, H), jnp.float32),
            pltpu.VMEM((_CK, H), jnp.float32),
            pltpu.SemaphoreType.DMA,
            pltpu.SemaphoreType.DMA,
            pltpu.VMEM_SHARED((n_nodes, H), jnp.float32),
        ],
    )
    def sk(val_hbm, dst_hbm, z_hbm, out_hbm, idx_v, rows_a, rows_b, sva, svb, acc):
        c = lax.axis_index("c")
        s = lax.axis_index("s")
        wid = s * NC + c
        rows = (rows_a, rows_b)
        sv = (sva, svb)

        @pl.when(s < NZ)
        def _():
            off = pl.multiple_of(s * _ZR, 8)
            pltpu.sync_copy(z_hbm, acc.at[pl.ds(off, _ZR), :])

        plsc.subcore_barrier()

        def body(j, carry):
            cs = (2 * j) * NW + wid
            for b in range(2):
                ck = cs + b * NW

                @pl.when(ck < nch)
                def _(ck=ck, b=b):
                    base = pl.multiple_of(ck * _CK, _CK)
                    pltpu.sync_copy(dst_hbm.at[pl.ds(base, _CK)], idx_v.at[b])
                    pltpu.async_copy(val_hbm.at[pl.ds(base, _CK), :], rows[b], sv[b])

            for b in range(2):
                ck = cs + b * NW

                @pl.when(ck < nch)
                def _(ck=ck, b=b):
                    base = pl.multiple_of(ck * _CK, _CK)
                    pltpu.make_async_copy(val_hbm.at[pl.ds(base, _CK), :], rows[b], sv[b]).wait()
                    pltpu.sync_copy(rows[b], acc.at[idx_v.at[b]], add=True)

            return carry

        lax.fori_loop(0, (iters + 1) // 2, body, 0)
        plsc.subcore_barrier()

        @pl.when(s < NZ)
        def _():
            off = pl.multiple_of(s * _ZR, 8)
            pltpu.sync_copy(acc.at[pl.ds(off, _ZR), :], out_hbm.at[c, pl.ds(off, _ZR), :])

    return sk(val, dst, zeros_blk)


def _embed(x, W_emb, b_emb, ln_g, ln_b):
    n_nodes, D = x.shape
    H = W_emb.shape[0]
    G = n_nodes // _BN

    def body(x_ref, w_ref, b_ref, g_ref, bb_ref, out_ref):
        xv = x_ref[...]
        hm = lax.dot_general(xv, w_ref[...], (((1,), (1,)), ((), ())),
                             preferred_element_type=jnp.float32) + b_ref[...]
        mu = jnp.mean(hm, axis=1, keepdims=True)
        var = jnp.mean((hm - mu) ** 2, axis=1, keepdims=True)
        hn = (hm - mu) / jnp.sqrt(var + 1e-5) * g_ref[...] + bb_ref[...]
        out_ref[...] = jnp.maximum(hn, 0.0)

    return pl.pallas_call(
        body,
        grid=(G,),
        in_specs=[
            pl.BlockSpec((_BN, D), lambda i: (i, 0)),
            pl.BlockSpec((H, D), lambda i: (0, 0)),
            pl.BlockSpec((1, H), lambda i: (0, 0)),
            pl.BlockSpec((1, H), lambda i: (0, 0)),
            pl.BlockSpec((1, H), lambda i: (0, 0)),
        ],
        out_specs=pl.BlockSpec((_BN, H), lambda i: (i, 0)),
        out_shape=jax.ShapeDtypeStruct((n_nodes, H), jnp.float32),
    )(x, W_emb, b_emb.reshape(1, H), ln_g.reshape(1, H), ln_b.reshape(1, H))


def _bspline_tr(hs, left, et_e, et_o, coeffs_ref, wb_ref, ws_ref, R):
    t0 = _KNOTS[0]
    inv_dt = 1.0 / (_KNOTS[1] - _KNOTS[0])
    base = hs * jax.nn.sigmoid(hs)
    u = (hs - t0) * inv_dt
    ifl = jnp.floor(u)
    f = u - ifl
    ii = ifl.astype(jnp.int32)
    f2 = f * f
    f3 = f2 * f
    onemf = 1.0 - f
    w0 = onemf * onemf * onemf * (1.0 / 6.0)
    w1 = 0.5 * f3 - f2 + (2.0 / 3.0)
    w2 = -0.5 * f3 + 0.5 * f2 + 0.5 * f + (1.0 / 6.0)
    w3 = f3 * (1.0 / 6.0)
    etw = jnp.where(left, et_e, et_o)
    ohf = [(etw == r).astype(hs.dtype) for r in range(R)]

    def _mix(vals_by_r):
        acc = ohf[0] * vals_by_r[0]
        for r in range(1, R):
            acc = acc + ohf[r] * vals_by_r[r]
        return acc

    ce = [_mix([coeffs_ref[r, n] for r in range(R)]) for n in range(_NB)]
    em = [ii == m for m in range(_NB + _DEG)]
    spline = jnp.zeros_like(hs)
    for k, w in enumerate((w0, w1, w2, w3)):
        cej = jnp.zeros_like(hs)
        for n in range(_NB):
            cej = jnp.where(em[n + 3 - k], ce[n], cej)
        spline = spline + w * cej
    wb = _mix([wb_ref[r] for r in range(R)])
    ws = _mix([ws_ref[r] for r in range(R)])
    return wb * base + ws * spline


def _edge_transform(hs2, etp, coeffs, w_base, w_spline, attention):
    EP, W = hs2.shape
    H = W // 2
    G = EP // _BE
    R = w_base.shape[0]

    def body(hs_ref, et_ref, coeffs_ref, wb_ref, ws_ref, att_ref, tr_ref, sc_ref):
        hs_v = hs_ref[...]
        etpair = et_ref[0]
        et_e = etpair[:, 0:1]
        et_o = etpair[:, 1:2]
        lane = lax.broadcasted_iota(jnp.int32, (_BE, W), 1)
        left = lane < H
        tr = _bspline_tr(hs_v, left, et_e, et_o, coeffs_ref, wb_ref, ws_ref, R)
        trw = tr * att_ref[...]
        half = (lax.broadcasted_iota(jnp.int32, (W, 2), 0) // H
                == lax.broadcasted_iota(jnp.int32, (W, 2), 1)).astype(jnp.float32)
        sc = lax.dot_general(trw, half, (((1,), (0,)), ((), ())),
                             preferred_element_type=jnp.float32)
        tr_ref[...] = tr
        sc_ref[0] = sc

    att2 = jnp.concatenate([attention, attention]).reshape(1, W)
    return pl.pallas_call(
        body,
        grid=(G,),
        in_specs=[
            pl.BlockSpec((_BE, W), lambda i: (i, 0)),
            pl.BlockSpec((1, _BE, 2), lambda i: (i, 0, 0)),
            pl.BlockSpec(memory_space=pltpu.SMEM),
            pl.BlockSpec(memory_space=pltpu.SMEM),
            pl.BlockSpec(memory_space=pltpu.SMEM),
            pl.BlockSpec((1, W), lambda i: (0, 0)),
        ],
        out_specs=[
            pl.BlockSpec((_BE, W), lambda i: (i, 0)),
            pl.BlockSpec((1, _BE, 2), lambda i: (i, 0, 0)),
        ],
        out_shape=[
            jax.ShapeDtypeStruct((EP, W), jnp.float32),
            jax.ShapeDtypeStruct((G, _BE, 2), jnp.float32),
        ],
    )(hs2, etp, coeffs, w_base, w_spline, att2)


def _scale(tr2, sc3, etp, sc_d, et_d):
    EP, W = tr2.shape
    H = W // 2
    G = EP // _BE
    R = 4

    def body(tr_ref, sc_ref, et_ref, scd_ref, etd_ref, val_ref, attn_ref, m_scr, s_scr):
        i = pl.program_id(0)

        @pl.when(i == 0)
        def _():
            scd = scd_ref[...]
            etd = etd_ref[...]
            for r in range(R):
                scm = jnp.where(etd == r, scd, -1e30)
                mr = jnp.max(scm)
                m_scr[r] = mr
                s_scr[r] = jnp.sum(jnp.exp(scm - mr))

        sc = sc_ref[0]
        et = et_ref[0]
        m_e = jnp.zeros_like(sc)
        s_e = jnp.ones_like(sc)
        for r in range(R):
            m_e = jnp.where(et == r, m_scr[r], m_e)
            s_e = jnp.where(et == r, s_scr[r], s_e)
        attn = jnp.exp(sc - m_e) / s_e
        lane = lax.broadcasted_iota(jnp.int32, (_BE, W), 1)
        attn_wide = jnp.where(lane < H, attn[:, 0:1], attn[:, 1:2])
        val_ref[...] = tr_ref[...] * attn_wide
        attn_ref[0] = attn

    return pl.pallas_call(
        body,
        grid=(G,),
        in_specs=[
            pl.BlockSpec((_BE, W), lambda i: (i, 0)),
            pl.BlockSpec((1, _BE, 2), lambda i: (i, 0, 0)),
            pl.BlockSpec((1, _BE, 2), lambda i: (i, 0, 0)),
            pl.BlockSpec(sc_d.shape, lambda i: (0, 0)),
            pl.BlockSpec(et_d.shape, lambda i: (0, 0)),
        ],
        out_specs=[
            pl.BlockSpec((_BE, W), lambda i: (i, 0)),
            pl.BlockSpec((1, _BE, 2), lambda i: (i, 0, 0)),
        ],
        out_shape=[
            jax.ShapeDtypeStruct((EP, W), jnp.float32),
            jax.ShapeDtypeStruct((G, _BE, 2), jnp.float32),
        ],
        scratch_shapes=[
            pltpu.SMEM((R,), jnp.float32),
            pltpu.SMEM((R,), jnp.float32),
        ],
    )(tr2, sc3, etp, sc_d, et_d)


def _gru(msg2, h, W_ih, W_hh, b_ih, b_hh):
    n_nodes, H = h.shape
    NC = msg2.shape[0]
    G = n_nodes // _BN

    def body(msg_ref, h_ref, wih_ref, whh_ref, bih_ref, bhh_ref, out_ref):
        msg = msg_ref[0]
        for c in range(1, NC):
            msg = msg + msg_ref[c]
        hv = h_ref[...]
        gi = lax.dot_general(msg, wih_ref[...], (((1,), (1,)), ((), ())),
                             preferred_element_type=jnp.float32) + bih_ref[...]
        gh = lax.dot_general(hv, whh_ref[...], (((1,), (1,)), ((), ())),
                             preferred_element_type=jnp.float32) + bhh_ref[...]
        rg = jax.nn.sigmoid(gi[:, :H] + gh[:, :H])
        zg = jax.nn.sigmoid(gi[:, H:2 * H] + gh[:, H:2 * H])
        ng = jnp.tanh(gi[:, 2 * H:] + rg * gh[:, 2 * H:])
        out_ref[...] = (1.0 - zg) * ng + zg * hv

    return pl.pallas_call(
        body,
        grid=(G,),
        in_specs=[
            pl.BlockSpec((NC, _BN, H), lambda i: (0, i, 0)),
            pl.BlockSpec((_BN, H), lambda i: (i, 0)),
            pl.BlockSpec((3 * H, H), lambda i: (0, 0)),
            pl.BlockSpec((3 * H, H), lambda i: (0, 0)),
            pl.BlockSpec((1, 3 * H), lambda i: (0, 0)),
            pl.BlockSpec((1, 3 * H), lambda i: (0, 0)),
        ],
        out_specs=pl.BlockSpec((_BN, H), lambda i: (i, 0)),
        out_shape=jax.ShapeDtypeStruct((n_nodes, H), jnp.float32),
    )(msg2, h, W_ih, W_hh, b_ih.reshape(1, 3 * H), b_hh.reshape(1, 3 * H))


def kernel(x, edge_index, edge_type, W_emb, b_emb, ln_g, ln_b, w_base, w_spline,
           coeffs, attention, W_ih, W_hh, b_ih, b_hh):
    n_nodes, _ = x.shape
    H = W_emb.shape[0]
    E = edge_type.shape[0]
    R = w_base.shape[0]
    src = edge_index[0].astype(jnp.int32)
    dst = edge_index[1].astype(jnp.int32)
    et = edge_type.astype(jnp.int32)
    etp = et.reshape(E // (2 * _BE), _BE, 2)
    et_d = et.reshape(E // 128, 128)
    zeros_blk = jnp.zeros((_ZR, H), jnp.float32)

    h = _embed(x, W_emb, b_emb, ln_g, ln_b)
    attns = []
    for _ in range(2):
        hs = _sc_gather(h, src)
        hs2 = hs.reshape(E // 2, 2 * H)
        tr2, sc3 = _edge_transform(hs2, etp, coeffs, w_base, w_spline, attention)
        val2, attn3 = _scale(tr2, sc3, etp, sc3.reshape(E // 128, 128), et_d)
        msg2 = _sc_scatter(val2.reshape(E, H), dst, zeros_blk, n_nodes)
        h = _gru(msg2, h, W_ih, W_hh, b_ih, b_hh)
        attns.append(attn3.reshape(E))
    return h, jnp.stack(attns)


# speedup vs baseline: 4.6303x; 1.0145x over previous
import functools

import numpy as np
import jax
import jax.numpy as jnp
from jax import lax
from jax.experimental import pallas as pl
from jax.experimental.pallas import tpu as pltpu
from jax.experimental.pallas import tpu_sc as plsc

_DEG = 3
_NB = 7
_KNOTS = [float(v) for v in np.linspace(-7.0, 7.0, _NB + _DEG + 1).astype(np.float32)]

_BE = 1000
_BN = 1000
_CK = 128
_ZR = 1000


def _sc_gather(h, src2):
    n_nodes, H = h.shape
    nch, _ = src2.shape
    E = nch * _CK
    info = plsc.get_sparse_core_info()
    NC, NS = info.num_cores, info.num_subcores
    NW = NC * NS
    base_cnt = nch // NW
    rem = nch - base_cnt * NW
    pairs = (base_cnt + 2) // 2
    mesh = plsc.VectorSubcoreMesh(core_axis_name="c", subcore_axis_name="s")

    @functools.partial(
        pl.kernel,
        out_type=jax.ShapeDtypeStruct((E, H), jnp.float32),
        mesh=mesh,
        compiler_params=pltpu.CompilerParams(use_tc_tiling_on_sc=False),
        scratch_types=[
            pltpu.VMEM((base_cnt + 1, _CK), jnp.int32),
            pltpu.VMEM((_CK, H), jnp.float32),
            pltpu.VMEM((_CK, H), jnp.float32),
            pltpu.SemaphoreType.DMA,
            pltpu.SemaphoreType.DMA,
            pltpu.SemaphoreType.DMA,
            pltpu.SemaphoreType.DMA,
        ],
    )
    def gk(h_hbm, src_hbm, out_hbm, idx_v, rows_a, rows_b, sga, sgb, swa, swb):
        wid = lax.axis_index("s") * NC + lax.axis_index("c")
        start = wid * base_cnt + jnp.minimum(wid, rem)
        cnt = base_cnt + jnp.where(wid < rem, 1, 0)
        rows = (rows_a, rows_b)
        sg = (sga, sgb)
        sw = (swa, swb)
        pltpu.sync_copy(src_hbm.at[pl.ds(start, base_cnt), :], idx_v.at[pl.ds(0, base_cnt), :])

        @pl.when(wid < rem)
        def _():
            pltpu.sync_copy(src_hbm.at[pl.ds(start + base_cnt, 1), :],
                            idx_v.at[pl.ds(base_cnt, 1), :])

        def body(j, carry):
            for b in range(2):
                slot = 2 * j + b

                @pl.when(slot < cnt)
                def _(slot=slot, b=b):
                    base = pl.multiple_of((start + slot) * _CK, _CK)

                    @pl.when(j > 0)
                    def _():
                        pltpu.make_async_copy(rows[b], out_hbm.at[pl.ds(base, _CK), :], sw[b]).wait()

                    pltpu.async_copy(h_hbm.at[idx_v.at[slot]], rows[b], sg[b])

            for b in range(2):
                slot = 2 * j + b

                @pl.when(slot < cnt)
                def _(slot=slot, b=b):
                    base = pl.multiple_of((start + slot) * _CK, _CK)
                    pltpu.make_async_copy(h_hbm.at[idx_v.at[slot]], rows[b], sg[b]).wait()
                    pltpu.async_copy(rows[b], out_hbm.at[pl.ds(base, _CK), :], sw[b])

            return carry

        lax.fori_loop(0, pairs, body, 0)
        for b in range(2):
            pltpu.make_async_copy(rows[b], out_hbm.at[pl.ds(0, _CK), :], sw[b]).wait()

    return gk(h, src2)


def _sc_scatter(val, dst2, zeros_blk, n_nodes):
    E, H = val.shape
    info = plsc.get_sparse_core_info()
    NC, NS = info.num_cores, info.num_subcores
    NW = NC * NS
    nch = E // _CK
    base_cnt = nch // NW
    rem = nch - base_cnt * NW
    pairs = (base_cnt + 2) // 2
    NZ = n_nodes // _ZR
    mesh = plsc.VectorSubcoreMesh(core_axis_name="c", subcore_axis_name="s")

    @functools.partial(
        pl.kernel,
        out_type=jax.ShapeDtypeStruct((NC, n_nodes, H), jnp.float32),
        mesh=mesh,
        compiler_params=pltpu.CompilerParams(use_tc_tiling_on_sc=False),
        scratch_types=[
            pltpu.VMEM((base_cnt + 1, _CK), jnp.int32),
            pltpu.VMEM((_CK, H), jnp.float32),
            pltpu.VMEM((_CK, H), jnp.float32),
            pltpu.SemaphoreType.DMA,
            pltpu.SemaphoreType.DMA,
            pltpu.VMEM_SHARED((n_nodes, H), jnp.float32),
        ],
    )
    def sk(val_hbm, dst_hbm, z_hbm, out_hbm, idx_v, rows_a, rows_b, sva, svb, acc):
        c = lax.axis_index("c")
        s = lax.axis_index("s")
        wid = s * NC + c
        start = wid * base_cnt + jnp.minimum(wid, rem)
        cnt = base_cnt + jnp.where(wid < rem, 1, 0)
        rows = (rows_a, rows_b)
        sv = (sva, svb)

        @pl.when(s < NZ)
        def _():
            off = pl.multiple_of(s * _ZR, 8)
            pltpu.sync_copy(z_hbm, acc.at[pl.ds(off, _ZR), :])

        pltpu.sync_copy(dst_hbm.at[pl.ds(start, base_cnt), :], idx_v.at[pl.ds(0, base_cnt), :])

        @pl.when(wid < rem)
        def _():
            pltpu.sync_copy(dst_hbm.at[pl.ds(start + base_cnt, 1), :],
                            idx_v.at[pl.ds(base_cnt, 1), :])

        plsc.subcore_barrier()

        def body(j, carry):
            for b in range(2):
                slot = 2 * j + b

                @pl.when(slot < cnt)
                def _(slot=slot, b=b):
                    base = pl.multiple_of((start + slot) * _CK, _CK)
                    pltpu.async_copy(val_hbm.at[pl.ds(base, _CK), :], rows[b], sv[b])

            for b in range(2):
                slot = 2 * j + b

                @pl.when(slot < cnt)
                def _(slot=slot, b=b):
                    base = pl.multiple_of((start + slot) * _CK, _CK)
                    pltpu.make_async_copy(val_hbm.at[pl.ds(base, _CK), :], rows[b], sv[b]).wait()
                    pltpu.sync_copy(rows[b], acc.at[idx_v.at[slot]], add=True)

            return carry

        lax.fori_loop(0, pairs, body, 0)
        plsc.subcore_barrier()

        @pl.when(s < NZ)
        def _():
            off = pl.multiple_of(s * _ZR, 8)
            pltpu.sync_copy(acc.at[pl.ds(off, _ZR), :], out_hbm.at[c, pl.ds(off, _ZR), :])

    return sk(val, dst2, zeros_blk)


def _embed(x, W_emb, b_emb, ln_g, ln_b):
    n_nodes, D = x.shape
    H = W_emb.shape[0]
    G = n_nodes // _BN

    def body(x_ref, w_ref, b_ref, g_ref, bb_ref, out_ref):
        xv = x_ref[...]
        hm = lax.dot_general(xv, w_ref[...], (((1,), (1,)), ((), ())),
                             preferred_element_type=jnp.float32) + b_ref[...]
        mu = jnp.mean(hm, axis=1, keepdims=True)
        var = jnp.mean((hm - mu) ** 2, axis=1, keepdims=True)
        hn = (hm - mu) / jnp.sqrt(var + 1e-5) * g_ref[...] + bb_ref[...]
        out_ref[...] = jnp.maximum(hn, 0.0)

    return pl.pallas_call(
        body,
        grid=(G,),
        in_specs=[
            pl.BlockSpec((_BN, D), lambda i: (i, 0)),
            pl.BlockSpec((H, D), lambda i: (0, 0)),
            pl.BlockSpec((1, H), lambda i: (0, 0)),
            pl.BlockSpec((1, H), lambda i: (0, 0)),
            pl.BlockSpec((1, H), lambda i: (0, 0)),
        ],
        out_specs=pl.BlockSpec((_BN, H), lambda i: (i, 0)),
        out_shape=jax.ShapeDtypeStruct((n_nodes, H), jnp.float32),
    )(x, W_emb, b_emb.reshape(1, H), ln_g.reshape(1, H), ln_b.reshape(1, H))


def _bspline_tr(hs, left, et_e, et_o, coeffs_ref, wb_ref, ws_ref, R):
    t0 = _KNOTS[0]
    inv_dt = 1.0 / (_KNOTS[1] - _KNOTS[0])
    base = hs * jax.nn.sigmoid(hs)
    u = (hs - t0) * inv_dt
    ifl = jnp.floor(u)
    f = u - ifl
    ii = ifl.astype(jnp.int32)
    f2 = f * f
    f3 = f2 * f
    onemf = 1.0 - f
    w0 = onemf * onemf * onemf * (1.0 / 6.0)
    w1 = 0.5 * f3 - f2 + (2.0 / 3.0)
    w2 = -0.5 * f3 + 0.5 * f2 + 0.5 * f + (1.0 / 6.0)
    w3 = f3 * (1.0 / 6.0)
    etw = jnp.where(left, et_e, et_o)
    ohf = [(etw == r).astype(hs.dtype) for r in range(R)]

    def _mix(vals_by_r):
        acc = ohf[0] * vals_by_r[0]
        for r in range(1, R):
            acc = acc + ohf[r] * vals_by_r[r]
        return acc

    ce = [_mix([coeffs_ref[r, n] for r in range(R)]) for n in range(_NB)]
    em = [ii == m for m in range(_NB + _DEG)]
    spline = jnp.zeros_like(hs)
    for k, w in enumerate((w0, w1, w2, w3)):
        cej = jnp.zeros_like(hs)
        for n in range(_NB):
            cej = jnp.where(em[n + 3 - k], ce[n], cej)
        spline = spline + w * cej
    wb = _mix([wb_ref[r] for r in range(R)])
    ws = _mix([ws_ref[r] for r in range(R)])
    return wb * base + ws * spline


def _edge_transform(hs2, etp, coeffs, w_base, w_spline, attention):
    EP, W = hs2.shape
    H = W // 2
    G = EP // _BE
    R = w_base.shape[0]

    def body(hs_ref, et_ref, coeffs_ref, wb_ref, ws_ref, att_ref, tr_ref, sc_ref):
        hs_v = hs_ref[...]
        etpair = et_ref[0]
        et_e = etpair[:, 0:1]
        et_o = etpair[:, 1:2]
        lane = lax.broadcasted_iota(jnp.int32, (_BE, W), 1)
        left = lane < H
        tr = _bspline_tr(hs_v, left, et_e, et_o, coeffs_ref, wb_ref, ws_ref, R)
        trw = tr * att_ref[...]
        half = (lax.broadcasted_iota(jnp.int32, (W, 2), 0) // H
                == lax.broadcasted_iota(jnp.int32, (W, 2), 1)).astype(jnp.float32)
        sc = lax.dot_general(trw, half, (((1,), (0,)), ((), ())),
                             preferred_element_type=jnp.float32)
        tr_ref[...] = tr
        sc_ref[0] = sc

    att2 = jnp.concatenate([attention, attention]).reshape(1, W)
    return pl.pallas_call(
        body,
        grid=(G,),
        in_specs=[
            pl.BlockSpec((_BE, W), lambda i: (i, 0)),
            pl.BlockSpec((1, _BE, 2), lambda i: (i, 0, 0)),
            pl.BlockSpec(memory_space=pltpu.SMEM),
            pl.BlockSpec(memory_space=pltpu.SMEM),
            pl.BlockSpec(memory_space=pltpu.SMEM),
            pl.BlockSpec((1, W), lambda i: (0, 0)),
        ],
        out_specs=[
            pl.BlockSpec((_BE, W), lambda i: (i, 0)),
            pl.BlockSpec((1, _BE, 2), lambda i: (i, 0, 0)),
        ],
        out_shape=[
            jax.ShapeDtypeStruct((EP, W), jnp.float32),
            jax.ShapeDtypeStruct((G, _BE, 2), jnp.float32),
        ],
    )(hs2, etp, coeffs, w_base, w_spline, att2)


def _scale(tr2, sc3, etp, sc_d, et_d):
    EP, W = tr2.shape
    H = W // 2
    G = EP // _BE
    R = 4

    def body(tr_ref, sc_ref, et_ref, scd_ref, etd_ref, val_ref, attn_ref, m_scr, s_scr):
        i = pl.program_id(0)

        @pl.when(i == 0)
        def _():
            scd = scd_ref[...]
            etd = etd_ref[...]
            for r in range(R):
                scm = jnp.where(etd == r, scd, -1e30)
                mr = jnp.max(scm)
                m_scr[r] = mr
                s_scr[r] = jnp.sum(jnp.exp(scm - mr))

        sc = sc_ref[0]
        et = et_ref[0]
        m_e = jnp.zeros_like(sc)
        s_e = jnp.ones_like(sc)
        for r in range(R):
            m_e = jnp.where(et == r, m_scr[r], m_e)
            s_e = jnp.where(et == r, s_scr[r], s_e)
        attn = jnp.exp(sc - m_e) / s_e
        lane = lax.broadcasted_iota(jnp.int32, (_BE, W), 1)
        attn_wide = jnp.where(lane < H, attn[:, 0:1], attn[:, 1:2])
        val_ref[...] = tr_ref[...] * attn_wide
        attn_ref[0] = attn

    return pl.pallas_call(
        body,
        grid=(G,),
        in_specs=[
            pl.BlockSpec((_BE, W), lambda i: (i, 0)),
            pl.BlockSpec((1, _BE, 2), lambda i: (i, 0, 0)),
            pl.BlockSpec((1, _BE, 2), lambda i: (i, 0, 0)),
            pl.BlockSpec(sc_d.shape, lambda i: (0, 0)),
            pl.BlockSpec(et_d.shape, lambda i: (0, 0)),
        ],
        out_specs=[
            pl.BlockSpec((_BE, W), lambda i: (i, 0)),
            pl.BlockSpec((1, _BE, 2), lambda i: (i, 0, 0)),
        ],
        out_shape=[
            jax.ShapeDtypeStruct((EP, W), jnp.float32),
            jax.ShapeDtypeStruct((G, _BE, 2), jnp.float32),
        ],
        scratch_shapes=[
            pltpu.SMEM((R,), jnp.float32),
            pltpu.SMEM((R,), jnp.float32),
        ],
    )(tr2, sc3, etp, sc_d, et_d)


def _gru(msg2, h, W_ih, W_hh, b_ih, b_hh):
    n_nodes, H = h.shape
    NC = msg2.shape[0]
    G = n_nodes // _BN

    def body(msg_ref, h_ref, wih_ref, whh_ref, bih_ref, bhh_ref, out_ref):
        msg = msg_ref[0]
        for c in range(1, NC):
            msg = msg + msg_ref[c]
        hv = h_ref[...]
        gi = lax.dot_general(msg, wih_ref[...], (((1,), (1,)), ((), ())),
                             preferred_element_type=jnp.float32) + bih_ref[...]
        gh = lax.dot_general(hv, whh_ref[...], (((1,), (1,)), ((), ())),
                             preferred_element_type=jnp.float32) + bhh_ref[...]
        rg = jax.nn.sigmoid(gi[:, :H] + gh[:, :H])
        zg = jax.nn.sigmoid(gi[:, H:2 * H] + gh[:, H:2 * H])
        ng = jnp.tanh(gi[:, 2 * H:] + rg * gh[:, 2 * H:])
        out_ref[...] = (1.0 - zg) * ng + zg * hv

    return pl.pallas_call(
        body,
        grid=(G,),
        in_specs=[
            pl.BlockSpec((NC, _BN, H), lambda i: (0, i, 0)),
            pl.BlockSpec((_BN, H), lambda i: (i, 0)),
            pl.BlockSpec((3 * H, H), lambda i: (0, 0)),
            pl.BlockSpec((3 * H, H), lambda i: (0, 0)),
            pl.BlockSpec((1, 3 * H), lambda i: (0, 0)),
            pl.BlockSpec((1, 3 * H), lambda i: (0, 0)),
        ],
        out_specs=pl.BlockSpec((_BN, H), lambda i: (i, 0)),
        out_shape=jax.ShapeDtypeStruct((n_nodes, H), jnp.float32),
    )(msg2, h, W_ih, W_hh, b_ih.reshape(1, 3 * H), b_hh.reshape(1, 3 * H))


def kernel(x, edge_index, edge_type, W_emb, b_emb, ln_g, ln_b, w_base, w_spline,
           coeffs, attention, W_ih, W_hh, b_ih, b_hh):
    n_nodes, _ = x.shape
    H = W_emb.shape[0]
    E = edge_type.shape[0]
    R = w_base.shape[0]
    src2 = edge_index[0].astype(jnp.int32).reshape(E // _CK, _CK)
    dst2 = edge_index[1].astype(jnp.int32).reshape(E // _CK, _CK)
    et = edge_type.astype(jnp.int32)
    etp = et.reshape(E // (2 * _BE), _BE, 2)
    et_d = et.reshape(E // 128, 128)
    zeros_blk = jnp.zeros((_ZR, H), jnp.float32)

    h = _embed(x, W_emb, b_emb, ln_g, ln_b)
    attns = []
    for _ in range(2):
        hs = _sc_gather(h, src2)
        hs2 = hs.reshape(E // 2, 2 * H)
        tr2, sc3 = _edge_transform(hs2, etp, coeffs, w_base, w_spline, attention)
        val2, attn3 = _scale(tr2, sc3, etp, sc3.reshape(E // 128, 128), et_d)
        msg2 = _sc_scatter(val2.reshape(E, H), dst2, zeros_blk, n_nodes)
        h = _gru(msg2, h, W_ih, W_hh, b_ih, b_hh)
        attns.append(attn3.reshape(E))
    return h, jnp.stack(attns)


# speedup vs baseline: 4.6872x; 1.0123x over previous
import functools

import numpy as np
import jax
import jax.numpy as jnp
from jax import lax
from jax.experimental import pallas as pl
from jax.experimental.pallas import tpu as pltpu
from jax.experimental.pallas import tpu_sc as plsc

_DEG = 3
_NB = 7
_KNOTS = [float(v) for v in np.linspace(-7.0, 7.0, _NB + _DEG + 1).astype(np.float32)]

_BE = 1000
_BN = 1000
_CK = 128
_ZR = 1000


def _sc_gather(h, src2):
    n_nodes, H = h.shape
    nch, _ = src2.shape
    E = nch * _CK
    info = plsc.get_sparse_core_info()
    NC, NS = info.num_cores, info.num_subcores
    NW = NC * NS
    base_cnt = nch // NW
    rem = nch - base_cnt * NW
    pairs = (base_cnt + 2) // 2
    mesh = plsc.VectorSubcoreMesh(core_axis_name="c", subcore_axis_name="s")

    @functools.partial(
        pl.kernel,
        out_type=jax.ShapeDtypeStruct((E, H), jnp.float32),
        mesh=mesh,
        compiler_params=pltpu.CompilerParams(use_tc_tiling_on_sc=False),
        scratch_types=[
            pltpu.VMEM((base_cnt + 1, _CK), jnp.int32),
            pltpu.VMEM((_CK, H), jnp.float32),
            pltpu.VMEM((_CK, H), jnp.float32),
            pltpu.SemaphoreType.DMA,
            pltpu.SemaphoreType.DMA,
            pltpu.SemaphoreType.DMA,
            pltpu.SemaphoreType.DMA,
        ],
    )
    def gk(h_hbm, src_hbm, out_hbm, idx_v, rows_a, rows_b, sga, sgb, swa, swb):
        wid = lax.axis_index("s") * NC + lax.axis_index("c")
        start = wid * base_cnt + jnp.minimum(wid, rem)
        cnt = base_cnt + jnp.where(wid < rem, 1, 0)
        rows = (rows_a, rows_b)
        sg = (sga, sgb)
        sw = (swa, swb)
        pltpu.sync_copy(src_hbm.at[pl.ds(start, base_cnt), :], idx_v.at[pl.ds(0, base_cnt), :])

        @pl.when(wid < rem)
        def _():
            pltpu.sync_copy(src_hbm.at[pl.ds(start + base_cnt, 1), :],
                            idx_v.at[pl.ds(base_cnt, 1), :])

        def body(j, carry):
            for b in range(2):
                slot = 2 * j + b

                @pl.when(slot < cnt)
                def _(slot=slot, b=b):
                    base = pl.multiple_of((start + slot) * _CK, _CK)

                    @pl.when(j > 0)
                    def _():
                        pltpu.make_async_copy(rows[b], out_hbm.at[pl.ds(base, _CK), :], sw[b]).wait()

                    pltpu.async_copy(h_hbm.at[idx_v.at[slot]], rows[b], sg[b])

            for b in range(2):
                slot = 2 * j + b

                @pl.when(slot < cnt)
                def _(slot=slot, b=b):
                    base = pl.multiple_of((start + slot) * _CK, _CK)
                    pltpu.make_async_copy(h_hbm.at[idx_v.at[slot]], rows[b], sg[b]).wait()
                    pltpu.async_copy(rows[b], out_hbm.at[pl.ds(base, _CK), :], sw[b])

            return carry

        lax.fori_loop(0, pairs, body, 0)
        for b in range(2):
            pltpu.make_async_copy(rows[b], out_hbm.at[pl.ds(0, _CK), :], sw[b]).wait()

    return gk(h, src2)


def _sc_scatter(val, dst2, zeros_blk, n_nodes):
    E, H = val.shape
    info = plsc.get_sparse_core_info()
    NC, NS = info.num_cores, info.num_subcores
    NW = NC * NS
    nch = E // _CK
    base_cnt = nch // NW
    rem = nch - base_cnt * NW
    pairs = (base_cnt + 2) // 2
    NZ = n_nodes // _ZR
    mesh = plsc.VectorSubcoreMesh(core_axis_name="c", subcore_axis_name="s")

    @functools.partial(
        pl.kernel,
        out_type=jax.ShapeDtypeStruct((NC, n_nodes, H), jnp.float32),
        mesh=mesh,
        compiler_params=pltpu.CompilerParams(use_tc_tiling_on_sc=False),
        scratch_types=[
            pltpu.VMEM((base_cnt + 1, _CK), jnp.int32),
            pltpu.VMEM((_CK, H), jnp.float32),
            pltpu.VMEM((_CK, H), jnp.float32),
            pltpu.SemaphoreType.DMA,
            pltpu.SemaphoreType.DMA,
            pltpu.VMEM_SHARED((n_nodes, H), jnp.float32),
        ],
    )
    def sk(val_hbm, dst_hbm, z_hbm, out_hbm, idx_v, rows_a, rows_b, sva, svb, acc):
        c = lax.axis_index("c")
        s = lax.axis_index("s")
        wid = s * NC + c
        start = wid * base_cnt + jnp.minimum(wid, rem)
        cnt = base_cnt + jnp.where(wid < rem, 1, 0)
        rows = (rows_a, rows_b)
        sv = (sva, svb)

        @pl.when(s < NZ)
        def _():
            off = pl.multiple_of(s * _ZR, 8)
            pltpu.sync_copy(z_hbm, acc.at[pl.ds(off, _ZR), :])

        pltpu.sync_copy(dst_hbm.at[pl.ds(start, base_cnt), :], idx_v.at[pl.ds(0, base_cnt), :])

        @pl.when(wid < rem)
        def _():
            pltpu.sync_copy(dst_hbm.at[pl.ds(start + base_cnt, 1), :],
                            idx_v.at[pl.ds(base_cnt, 1), :])

        plsc.subcore_barrier()

        def body(j, carry):
            for b in range(2):
                slot = 2 * j + b

                @pl.when(slot < cnt)
                def _(slot=slot, b=b):
                    base = pl.multiple_of((start + slot) * _CK, _CK)
                    pltpu.async_copy(val_hbm.at[pl.ds(base, _CK), :], rows[b], sv[b])

            for b in range(2):
                slot = 2 * j + b

                @pl.when(slot < cnt)
                def _(slot=slot, b=b):
                    base = pl.multiple_of((start + slot) * _CK, _CK)
                    pltpu.make_async_copy(val_hbm.at[pl.ds(base, _CK), :], rows[b], sv[b]).wait()
                    pltpu.sync_copy(rows[b], acc.at[idx_v.at[slot]], add=True)

            return carry

        lax.fori_loop(0, pairs, body, 0)
        plsc.subcore_barrier()

        @pl.when(s < NZ)
        def _():
            off = pl.multiple_of(s * _ZR, 8)
            pltpu.sync_copy(acc.at[pl.ds(off, _ZR), :], out_hbm.at[c, pl.ds(off, _ZR), :])

    return sk(val, dst2, zeros_blk)


def _embed(x, W_emb, b_emb, ln_g, ln_b):
    n_nodes, D = x.shape
    H = W_emb.shape[0]
    G = n_nodes // _BN

    def body(x_ref, w_ref, b_ref, g_ref, bb_ref, out_ref):
        xv = x_ref[...]
        hm = lax.dot_general(xv, w_ref[...], (((1,), (1,)), ((), ())),
                             preferred_element_type=jnp.float32) + b_ref[...]
        mu = jnp.mean(hm, axis=1, keepdims=True)
        var = jnp.mean((hm - mu) ** 2, axis=1, keepdims=True)
        hn = (hm - mu) / jnp.sqrt(var + 1e-5) * g_ref[...] + bb_ref[...]
        out_ref[...] = jnp.maximum(hn, 0.0)

    return pl.pallas_call(
        body,
        grid=(G,),
        in_specs=[
            pl.BlockSpec((_BN, D), lambda i: (i, 0)),
            pl.BlockSpec((H, D), lambda i: (0, 0)),
            pl.BlockSpec((1, H), lambda i: (0, 0)),
            pl.BlockSpec((1, H), lambda i: (0, 0)),
            pl.BlockSpec((1, H), lambda i: (0, 0)),
        ],
        out_specs=pl.BlockSpec((_BN, H), lambda i: (i, 0)),
        out_shape=jax.ShapeDtypeStruct((n_nodes, H), jnp.float32),
    )(x, W_emb, b_emb.reshape(1, H), ln_g.reshape(1, H), ln_b.reshape(1, H))


def _bspline_tr(hs, left, et_e, et_o, coeffs_ref, wb_ref, ws_ref, R):
    t0 = _KNOTS[0]
    inv_dt = 1.0 / (_KNOTS[1] - _KNOTS[0])
    base = hs * jax.nn.sigmoid(hs)
    u = (hs - t0) * inv_dt
    ifl = jnp.floor(u)
    f = u - ifl
    ii = ifl.astype(jnp.int32)
    f2 = f * f
    f3 = f2 * f
    onemf = 1.0 - f
    w0 = onemf * onemf * onemf * (1.0 / 6.0)
    w1 = 0.5 * f3 - f2 + (2.0 / 3.0)
    w2 = -0.5 * f3 + 0.5 * f2 + 0.5 * f + (1.0 / 6.0)
    w3 = f3 * (1.0 / 6.0)
    etw = jnp.where(left, et_e, et_o)
    ohf = [(etw == r).astype(hs.dtype) for r in range(R)]

    def _mix(vals_by_r):
        acc = ohf[0] * vals_by_r[0]
        for r in range(1, R):
            acc = acc + ohf[r] * vals_by_r[r]
        return acc

    ce = [_mix([coeffs_ref[r, n] for r in range(R)]) for n in range(_NB)]
    em = [ii == m for m in range(_NB + _DEG)]
    spline = jnp.zeros_like(hs)
    for k, w in enumerate((w0, w1, w2, w3)):
        cej = jnp.zeros_like(hs)
        for n in range(_NB):
            cej = jnp.where(em[n + 3 - k], ce[n], cej)
        spline = spline + w * cej
    wb = _mix([wb_ref[r] for r in range(R)])
    ws = _mix([ws_ref[r] for r in range(R)])
    return wb * base + ws * spline


def _edge_transform(hs2, etp, coeffs, w_base, w_spline, attention):
    EP, W = hs2.shape
    H = W // 2
    G = EP // _BE
    R = w_base.shape[0]

    def body(hs_ref, et_ref, coeffs_ref, wb_ref, ws_ref, att_ref, tr_ref, sc_ref):
        hs_v = hs_ref[...]
        etpair = et_ref[0]
        et_e = etpair[:, 0:1]
        et_o = etpair[:, 1:2]
        lane = lax.broadcasted_iota(jnp.int32, (_BE, W), 1)
        left = lane < H
        tr = _bspline_tr(hs_v, left, et_e, et_o, coeffs_ref, wb_ref, ws_ref, R)
        trw = tr * att_ref[...]
        half = (lax.broadcasted_iota(jnp.int32, (W, 2), 0) // H
                == lax.broadcasted_iota(jnp.int32, (W, 2), 1)).astype(jnp.float32)
        sc = lax.dot_general(trw, half, (((1,), (0,)), ((), ())),
                             precision=lax.Precision.HIGHEST,
                             preferred_element_type=jnp.float32)
        tr_ref[...] = tr
        sc_ref[0] = sc

    att2 = jnp.concatenate([attention, attention]).reshape(1, W)
    return pl.pallas_call(
        body,
        grid=(G,),
        in_specs=[
            pl.BlockSpec((_BE, W), lambda i: (i, 0)),
            pl.BlockSpec((1, _BE, 2), lambda i: (i, 0, 0)),
            pl.BlockSpec(memory_space=pltpu.SMEM),
            pl.BlockSpec(memory_space=pltpu.SMEM),
            pl.BlockSpec(memory_space=pltpu.SMEM),
            pl.BlockSpec((1, W), lambda i: (0, 0)),
        ],
        out_specs=[
            pl.BlockSpec((_BE, W), lambda i: (i, 0)),
            pl.BlockSpec((1, _BE, 2), lambda i: (i, 0, 0)),
        ],
        out_shape=[
            jax.ShapeDtypeStruct((EP, W), jnp.float32),
            jax.ShapeDtypeStruct((G, _BE, 2), jnp.float32),
        ],
    )(hs2, etp, coeffs, w_base, w_spline, att2)


def _scale(tr2, sc3, etp, sc_d, et_d):
    EP, W = tr2.shape
    H = W // 2
    G = EP // _BE
    R = 4

    def body(tr_ref, sc_ref, et_ref, scd_ref, etd_ref, val_ref, attn_ref, m_scr, s_scr):
        i = pl.program_id(0)

        @pl.when(i == 0)
        def _():
            scd = scd_ref[...]
            etd = etd_ref[...]
            for r in range(R):
                scm = jnp.where(etd == r, scd, -1e30)
                mr = jnp.max(scm)
                m_scr[r] = mr
                s_scr[r] = jnp.sum(jnp.exp(scm - mr))

        sc = sc_ref[0]
        et = et_ref[0]
        m_e = jnp.zeros_like(sc)
        s_e = jnp.ones_like(sc)
        for r in range(R):
            m_e = jnp.where(et == r, m_scr[r], m_e)
            s_e = jnp.where(et == r, s_scr[r], s_e)
        attn = jnp.exp(sc - m_e) / s_e
        lane = lax.broadcasted_iota(jnp.int32, (_BE, W), 1)
        attn_wide = jnp.where(lane < H, attn[:, 0:1], attn[:, 1:2])
        val_ref[...] = tr_ref[...] * attn_wide
        attn_ref[0] = attn

    return pl.pallas_call(
        body,
        grid=(G,),
        in_specs=[
            pl.BlockSpec((_BE, W), lambda i: (i, 0)),
            pl.BlockSpec((1, _BE, 2), lambda i: (i, 0, 0)),
            pl.BlockSpec((1, _BE, 2), lambda i: (i, 0, 0)),
            pl.BlockSpec(sc_d.shape, lambda i: (0, 0)),
            pl.BlockSpec(et_d.shape, lambda i: (0, 0)),
        ],
        out_specs=[
            pl.BlockSpec((_BE, W), lambda i: (i, 0)),
            pl.BlockSpec((1, _BE, 2), lambda i: (i, 0, 0)),
        ],
        out_shape=[
            jax.ShapeDtypeStruct((EP, W), jnp.float32),
            jax.ShapeDtypeStruct((G, _BE, 2), jnp.float32),
        ],
        scratch_shapes=[
            pltpu.SMEM((R,), jnp.float32),
            pltpu.SMEM((R,), jnp.float32),
        ],
    )(tr2, sc3, etp, sc_d, et_d)


def _gru(msg2, h, W_ih, W_hh, b_ih, b_hh):
    n_nodes, H = h.shape
    NC = msg2.shape[0]
    G = n_nodes // _BN

    def body(msg_ref, h_ref, wih_ref, whh_ref, bih_ref, bhh_ref, out_ref):
        msg = msg_ref[0]
        for c in range(1, NC):
            msg = msg + msg_ref[c]
        hv = h_ref[...]
        gi = lax.dot_general(msg, wih_ref[...], (((1,), (1,)), ((), ())),
                             preferred_element_type=jnp.float32) + bih_ref[...]
        gh = lax.dot_general(hv, whh_ref[...], (((1,), (1,)), ((), ())),
                             preferred_element_type=jnp.float32) + bhh_ref[...]
        rg = jax.nn.sigmoid(gi[:, :H] + gh[:, :H])
        zg = jax.nn.sigmoid(gi[:, H:2 * H] + gh[:, H:2 * H])
        ng = jnp.tanh(gi[:, 2 * H:] + rg * gh[:, 2 * H:])
        out_ref[...] = (1.0 - zg) * ng + zg * hv

    return pl.pallas_call(
        body,
        grid=(G,),
        in_specs=[
            pl.BlockSpec((NC, _BN, H), lambda i: (0, i, 0)),
            pl.BlockSpec((_BN, H), lambda i: (i, 0)),
            pl.BlockSpec((3 * H, H), lambda i: (0, 0)),
            pl.BlockSpec((3 * H, H), lambda i: (0, 0)),
            pl.BlockSpec((1, 3 * H), lambda i: (0, 0)),
            pl.BlockSpec((1, 3 * H), lambda i: (0, 0)),
        ],
        out_specs=pl.BlockSpec((_BN, H), lambda i: (i, 0)),
        out_shape=jax.ShapeDtypeStruct((n_nodes, H), jnp.float32),
    )(msg2, h, W_ih, W_hh, b_ih.reshape(1, 3 * H), b_hh.reshape(1, 3 * H))


def kernel(x, edge_index, edge_type, W_emb, b_emb, ln_g, ln_b, w_base, w_spline,
           coeffs, attention, W_ih, W_hh, b_ih, b_hh):
    n_nodes, _ = x.shape
    H = W_emb.shape[0]
    E = edge_type.shape[0]
    R = w_base.shape[0]
    src2 = edge_index[0].astype(jnp.int32).reshape(E // _CK, _CK)
    dst2 = edge_index[1].astype(jnp.int32).reshape(E // _CK, _CK)
    et = edge_type.astype(jnp.int32)
    etp = et.reshape(E // (2 * _BE), _BE, 2)
    et_d = et.reshape(E // 128, 128)
    zeros_blk = jnp.zeros((_ZR, H), jnp.float32)

    h = _embed(x, W_emb, b_emb, ln_g, ln_b)
    attns = []
    for _ in range(2):
        hs = _sc_gather(h, src2)
        hs2 = hs.reshape(E // 2, 2 * H)
        tr2, sc3 = _edge_transform(hs2, etp, coeffs, w_base, w_spline, attention)
        val2, attn3 = _scale(tr2, sc3, etp, sc3.reshape(E // 128, 128), et_d)
        msg2 = _sc_scatter(val2.reshape(E, H), dst2, zeros_blk, n_nodes)
        h = _gru(msg2, h, W_ih, W_hh, b_ih, b_hh)
        attns.append(attn3.reshape(E))
    return h, jnp.stack(attns)


# speedup vs baseline: 4.6882x; 1.0002x over previous
import functools

import numpy as np
import jax
import jax.numpy as jnp
from jax import lax
from jax.experimental import pallas as pl
from jax.experimental.pallas import tpu as pltpu
from jax.experimental.pallas import tpu_sc as plsc

_DEG = 3
_NB = 7
_KNOTS = [float(v) for v in np.linspace(-7.0, 7.0, _NB + _DEG + 1).astype(np.float32)]

_BE = 1000
_BN = 1000
_CK = 128
_ZR = 1000


def _sc_gather(h, src2):
    n_nodes, H = h.shape
    nch, _ = src2.shape
    E = nch * _CK
    info = plsc.get_sparse_core_info()
    NC, NS = info.num_cores, info.num_subcores
    NW = NC * NS
    base_cnt = nch // NW
    rem = nch - base_cnt * NW
    pairs = (base_cnt + 2) // 2
    mesh = plsc.VectorSubcoreMesh(core_axis_name="c", subcore_axis_name="s")

    @functools.partial(
        pl.kernel,
        out_type=jax.ShapeDtypeStruct((E, H), jnp.float32),
        mesh=mesh,
        compiler_params=pltpu.CompilerParams(use_tc_tiling_on_sc=False),
        scratch_types=[
            pltpu.VMEM((base_cnt + 1, _CK), jnp.int32),
            pltpu.VMEM((_CK, H), jnp.float32),
            pltpu.VMEM((_CK, H), jnp.float32),
            pltpu.SemaphoreType.DMA,
            pltpu.SemaphoreType.DMA,
            pltpu.SemaphoreType.DMA,
            pltpu.SemaphoreType.DMA,
        ],
    )
    def gk(h_hbm, src_hbm, out_hbm, idx_v, rows_a, rows_b, sga, sgb, swa, swb):
        wid = lax.axis_index("s") * NC + lax.axis_index("c")
        start = wid * base_cnt + jnp.minimum(wid, rem)
        cnt = base_cnt + jnp.where(wid < rem, 1, 0)
        rows = (rows_a, rows_b)
        sg = (sga, sgb)
        sw = (swa, swb)
        pltpu.sync_copy(src_hbm.at[pl.ds(start, base_cnt), :], idx_v.at[pl.ds(0, base_cnt), :])

        @pl.when(wid < rem)
        def _():
            pltpu.sync_copy(src_hbm.at[pl.ds(start + base_cnt, 1), :],
                            idx_v.at[pl.ds(base_cnt, 1), :])

        def body(j, carry):
            for b in range(2):
                slot = 2 * j + b

                @pl.when(slot < cnt)
                def _(slot=slot, b=b):
                    base = pl.multiple_of((start + slot) * _CK, _CK)

                    @pl.when(j > 0)
                    def _():
                        pltpu.make_async_copy(rows[b], out_hbm.at[pl.ds(base, _CK), :], sw[b]).wait()

                    pltpu.async_copy(h_hbm.at[idx_v.at[slot]], rows[b], sg[b])

            for b in range(2):
                slot = 2 * j + b

                @pl.when(slot < cnt)
                def _(slot=slot, b=b):
                    base = pl.multiple_of((start + slot) * _CK, _CK)
                    pltpu.make_async_copy(h_hbm.at[idx_v.at[slot]], rows[b], sg[b]).wait()
                    pltpu.async_copy(rows[b], out_hbm.at[pl.ds(base, _CK), :], sw[b])

            return carry

        lax.fori_loop(0, pairs, body, 0)
        for b in range(2):
            pltpu.make_async_copy(rows[b], out_hbm.at[pl.ds(0, _CK), :], sw[b]).wait()

    return gk(h, src2)


def _sc_scatter(val, dst2, zeros_blk, n_nodes):
    E, H = val.shape
    info = plsc.get_sparse_core_info()
    NC, NS = info.num_cores, info.num_subcores
    NW = NC * NS
    nch = E // _CK
    base_cnt = nch // NW
    rem = nch - base_cnt * NW
    pairs = (base_cnt + 2) // 2
    NZ = n_nodes // _ZR
    mesh = plsc.VectorSubcoreMesh(core_axis_name="c", subcore_axis_name="s")

    @functools.partial(
        pl.kernel,
        out_type=jax.ShapeDtypeStruct((NC, n_nodes, H), jnp.float32),
        mesh=mesh,
        compiler_params=pltpu.CompilerParams(use_tc_tiling_on_sc=False),
        scratch_types=[
            pltpu.VMEM((base_cnt + 1, _CK), jnp.int32),
            pltpu.VMEM((_CK, H), jnp.float32),
            pltpu.VMEM((_CK, H), jnp.float32),
            pltpu.SemaphoreType.DMA,
            pltpu.SemaphoreType.DMA,
            pltpu.VMEM_SHARED((n_nodes, H), jnp.float32),
        ],
    )
    def sk(val_hbm, dst_hbm, z_hbm, out_hbm, idx_v, rows_a, rows_b, sva, svb, acc):
        c = lax.axis_index("c")
        s = lax.axis_index("s")
        wid = s * NC + c
        start = wid * base_cnt + jnp.minimum(wid, rem)
        cnt = base_cnt + jnp.where(wid < rem, 1, 0)
        rows = (rows_a, rows_b)
        sv = (sva, svb)

        @pl.when(s < NZ)
        def _():
            off = pl.multiple_of(s * _ZR, 8)
            pltpu.sync_copy(z_hbm, acc.at[pl.ds(off, _ZR), :])

        pltpu.sync_copy(dst_hbm.at[pl.ds(start, base_cnt), :], idx_v.at[pl.ds(0, base_cnt), :])

        @pl.when(wid < rem)
        def _():
            pltpu.sync_copy(dst_hbm.at[pl.ds(start + base_cnt, 1), :],
                            idx_v.at[pl.ds(base_cnt, 1), :])

        plsc.subcore_barrier()

        def body(j, carry):
            for b in range(2):
                slot = 2 * j + b

                @pl.when(slot < cnt)
                def _(slot=slot, b=b):
                    base = pl.multiple_of((start + slot) * _CK, _CK)
                    pltpu.async_copy(val_hbm.at[pl.ds(base, _CK), :], rows[b], sv[b])

            for b in range(2):
                slot = 2 * j + b

                @pl.when(slot < cnt)
                def _(slot=slot, b=b):
                    base = pl.multiple_of((start + slot) * _CK, _CK)
                    pltpu.make_async_copy(val_hbm.at[pl.ds(base, _CK), :], rows[b], sv[b]).wait()
                    pltpu.sync_copy(rows[b], acc.at[idx_v.at[slot]], add=True)

            return carry

        lax.fori_loop(0, pairs, body, 0)
        plsc.subcore_barrier()

        @pl.when(s < NZ)
        def _():
            off = pl.multiple_of(s * _ZR, 8)
            pltpu.sync_copy(acc.at[pl.ds(off, _ZR), :], out_hbm.at[c, pl.ds(off, _ZR), :])

    return sk(val, dst2, zeros_blk)


def _embed(x, W_emb, b_emb, ln_g, ln_b):
    n_nodes, D = x.shape
    H = W_emb.shape[0]
    G = n_nodes // _BN

    def body(x_ref, w_ref, b_ref, g_ref, bb_ref, out_ref):
        xv = x_ref[...]
        hm = lax.dot_general(xv, w_ref[...], (((1,), (1,)), ((), ())),
                             preferred_element_type=jnp.float32) + b_ref[...]
        mu = jnp.mean(hm, axis=1, keepdims=True)
        var = jnp.mean((hm - mu) ** 2, axis=1, keepdims=True)
        hn = (hm - mu) / jnp.sqrt(var + 1e-5) * g_ref[...] + bb_ref[...]
        out_ref[...] = jnp.maximum(hn, 0.0)

    return pl.pallas_call(
        body,
        grid=(G,),
        in_specs=[
            pl.BlockSpec((_BN, D), lambda i: (i, 0)),
            pl.BlockSpec((H, D), lambda i: (0, 0)),
            pl.BlockSpec((1, H), lambda i: (0, 0)),
            pl.BlockSpec((1, H), lambda i: (0, 0)),
            pl.BlockSpec((1, H), lambda i: (0, 0)),
        ],
        out_specs=pl.BlockSpec((_BN, H), lambda i: (i, 0)),
        out_shape=jax.ShapeDtypeStruct((n_nodes, H), jnp.float32),
    )(x, W_emb, b_emb.reshape(1, H), ln_g.reshape(1, H), ln_b.reshape(1, H))


def _bspline_tr(hs, left, et_e, et_o, coeffs_ref, wb_ref, ws_ref, R):
    t0 = _KNOTS[0]
    inv_dt = 1.0 / (_KNOTS[1] - _KNOTS[0])
    base = hs * jax.nn.sigmoid(hs)
    u = (hs - t0) * inv_dt
    ifl = jnp.floor(u)
    f = u - ifl
    ii = ifl.astype(jnp.int32)
    f2 = f * f
    f3 = f2 * f
    onemf = 1.0 - f
    w0 = onemf * onemf * onemf * (1.0 / 6.0)
    w1 = 0.5 * f3 - f2 + (2.0 / 3.0)
    w2 = -0.5 * f3 + 0.5 * f2 + 0.5 * f + (1.0 / 6.0)
    w3 = f3 * (1.0 / 6.0)
    etw = jnp.where(left, et_e, et_o)
    ohb = [etw == r for r in range(R - 1)]

    def _mix(vals_by_r):
        acc = jnp.broadcast_to(jnp.asarray(vals_by_r[R - 1], hs.dtype), hs.shape)
        for r in range(R - 1):
            acc = jnp.where(ohb[r], vals_by_r[r], acc)
        return acc

    ce = [_mix([coeffs_ref[r, n] for r in range(R)]) for n in range(_NB)]
    em = [ii == m for m in range(_NB + _DEG)]
    spline = jnp.zeros_like(hs)
    for k, w in enumerate((w0, w1, w2, w3)):
        cej = jnp.zeros_like(hs)
        for n in range(_NB):
            cej = jnp.where(em[n + 3 - k], ce[n], cej)
        spline = spline + w * cej
    wb = _mix([wb_ref[r] for r in range(R)])
    ws = _mix([ws_ref[r] for r in range(R)])
    return wb * base + ws * spline


def _edge_transform(hs2, etp, coeffs, w_base, w_spline, attention):
    EP, W = hs2.shape
    H = W // 2
    G = EP // _BE
    R = w_base.shape[0]

    def body(hs_ref, et_ref, coeffs_ref, wb_ref, ws_ref, att_ref, tr_ref, sc_ref):
        hs_v = hs_ref[...]
        etpair = et_ref[0]
        et_e = etpair[:, 0:1]
        et_o = etpair[:, 1:2]
        lane = lax.broadcasted_iota(jnp.int32, (_BE, W), 1)
        left = lane < H
        tr = _bspline_tr(hs_v, left, et_e, et_o, coeffs_ref, wb_ref, ws_ref, R)
        trw = tr * att_ref[...]
        half = (lax.broadcasted_iota(jnp.int32, (W, 2), 0) // H
                == lax.broadcasted_iota(jnp.int32, (W, 2), 1)).astype(jnp.float32)
        sc = lax.dot_general(trw, half, (((1,), (0,)), ((), ())),
                             precision=lax.Precision.HIGHEST,
                             preferred_element_type=jnp.float32)
        tr_ref[...] = tr
        sc_ref[0] = sc

    att2 = jnp.concatenate([attention, attention]).reshape(1, W)
    return pl.pallas_call(
        body,
        grid=(G,),
        in_specs=[
            pl.BlockSpec((_BE, W), lambda i: (i, 0)),
            pl.BlockSpec((1, _BE, 2), lambda i: (i, 0, 0)),
            pl.BlockSpec(memory_space=pltpu.SMEM),
            pl.BlockSpec(memory_space=pltpu.SMEM),
            pl.BlockSpec(memory_space=pltpu.SMEM),
            pl.BlockSpec((1, W), lambda i: (0, 0)),
        ],
        out_specs=[
            pl.BlockSpec((_BE, W), lambda i: (i, 0)),
            pl.BlockSpec((1, _BE, 2), lambda i: (i, 0, 0)),
        ],
        out_shape=[
            jax.ShapeDtypeStruct((EP, W), jnp.float32),
            jax.ShapeDtypeStruct((G, _BE, 2), jnp.float32),
        ],
    )(hs2, etp, coeffs, w_base, w_spline, att2)


def _scale(tr2, sc3, etp, sc_d, et_d):
    EP, W = tr2.shape
    H = W // 2
    G = EP // _BE
    R = 4

    def body(tr_ref, sc_ref, et_ref, scd_ref, etd_ref, val_ref, attn_ref, m_scr, s_scr):
        i = pl.program_id(0)

        @pl.when(i == 0)
        def _():
            scd = scd_ref[...]
            etd = etd_ref[...]
            for r in range(R):
                scm = jnp.where(etd == r, scd, -1e30)
                mr = jnp.max(scm)
                m_scr[r] = mr
                s_scr[r] = jnp.sum(jnp.exp(scm - mr))

        sc = sc_ref[0]
        et = et_ref[0]
        m_e = jnp.zeros_like(sc)
        s_e = jnp.ones_like(sc)
        for r in range(R):
            m_e = jnp.where(et == r, m_scr[r], m_e)
            s_e = jnp.where(et == r, s_scr[r], s_e)
        attn = jnp.exp(sc - m_e) / s_e
        lane = lax.broadcasted_iota(jnp.int32, (_BE, W), 1)
        attn_wide = jnp.where(lane < H, attn[:, 0:1], attn[:, 1:2])
        val_ref[...] = tr_ref[...] * attn_wide
        attn_ref[0] = attn

    return pl.pallas_call(
        body,
        grid=(G,),
        in_specs=[
            pl.BlockSpec((_BE, W), lambda i: (i, 0)),
            pl.BlockSpec((1, _BE, 2), lambda i: (i, 0, 0)),
            pl.BlockSpec((1, _BE, 2), lambda i: (i, 0, 0)),
            pl.BlockSpec(sc_d.shape, lambda i: (0, 0)),
            pl.BlockSpec(et_d.shape, lambda i: (0, 0)),
        ],
        out_specs=[
            pl.BlockSpec((_BE, W), lambda i: (i, 0)),
            pl.BlockSpec((1, _BE, 2), lambda i: (i, 0, 0)),
        ],
        out_shape=[
            jax.ShapeDtypeStruct((EP, W), jnp.float32),
            jax.ShapeDtypeStruct((G, _BE, 2), jnp.float32),
        ],
        scratch_shapes=[
            pltpu.SMEM((R,), jnp.float32),
            pltpu.SMEM((R,), jnp.float32),
        ],
    )(tr2, sc3, etp, sc_d, et_d)


def _gru(msg2, h, W_ih, W_hh, b_ih, b_hh):
    n_nodes, H = h.shape
    NC = msg2.shape[0]
    G = n_nodes // _BN

    def body(msg_ref, h_ref, wih_ref, whh_ref, bih_ref, bhh_ref, out_ref):
        msg = msg_ref[0]
        for c in range(1, NC):
            msg = msg + msg_ref[c]
        hv = h_ref[...]
        gi = lax.dot_general(msg, wih_ref[...], (((1,), (1,)), ((), ())),
                             preferred_element_type=jnp.float32) + bih_ref[...]
        gh = lax.dot_general(hv, whh_ref[...], (((1,), (1,)), ((), ())),
                             preferred_element_type=jnp.float32) + bhh_ref[...]
        rg = jax.nn.sigmoid(gi[:, :H] + gh[:, :H])
        zg = jax.nn.sigmoid(gi[:, H:2 * H] + gh[:, H:2 * H])
        ng = jnp.tanh(gi[:, 2 * H:] + rg * gh[:, 2 * H:])
        out_ref[...] = (1.0 - zg) * ng + zg * hv

    return pl.pallas_call(
        body,
        grid=(G,),
        in_specs=[
            pl.BlockSpec((NC, _BN, H), lambda i: (0, i, 0)),
            pl.BlockSpec((_BN, H), lambda i: (i, 0)),
            pl.BlockSpec((3 * H, H), lambda i: (0, 0)),
            pl.BlockSpec((3 * H, H), lambda i: (0, 0)),
            pl.BlockSpec((1, 3 * H), lambda i: (0, 0)),
            pl.BlockSpec((1, 3 * H), lambda i: (0, 0)),
        ],
        out_specs=pl.BlockSpec((_BN, H), lambda i: (i, 0)),
        out_shape=jax.ShapeDtypeStruct((n_nodes, H), jnp.float32),
    )(msg2, h, W_ih, W_hh, b_ih.reshape(1, 3 * H), b_hh.reshape(1, 3 * H))


def kernel(x, edge_index, edge_type, W_emb, b_emb, ln_g, ln_b, w_base, w_spline,
           coeffs, attention, W_ih, W_hh, b_ih, b_hh):
    n_nodes, _ = x.shape
    H = W_emb.shape[0]
    E = edge_type.shape[0]
    R = w_base.shape[0]
    src2 = edge_index[0].astype(jnp.int32).reshape(E // _CK, _CK)
    dst2 = edge_index[1].astype(jnp.int32).reshape(E // _CK, _CK)
    et = edge_type.astype(jnp.int32)
    etp = et.reshape(E // (2 * _BE), _BE, 2)
    et_d = et.reshape(E // 128, 128)
    zeros_blk = jnp.zeros((_ZR, H), jnp.float32)

    h = _embed(x, W_emb, b_emb, ln_g, ln_b)
    attns = []
    for _ in range(2):
        hs = _sc_gather(h, src2)
        hs2 = hs.reshape(E // 2, 2 * H)
        tr2, sc3 = _edge_transform(hs2, etp, coeffs, w_base, w_spline, attention)
        val2, attn3 = _scale(tr2, sc3, etp, sc3.reshape(E // 128, 128), et_d)
        msg2 = _sc_scatter(val2.reshape(E, H), dst2, zeros_blk, n_nodes)
        h = _gru(msg2, h, W_ih, W_hh, b_ih, b_hh)
        attns.append(attn3.reshape(E))
    return h, jnp.stack(attns)
